# baseline (speedup 1.0000x reference)
import functools

import jax
import jax.numpy as jnp
from jax.experimental import pallas as pl
from jax.experimental.pallas import tpu as pltpu

_PREC = jax.lax.Precision.HIGHEST
_CPREC = jax.lax.Precision.DEFAULT
_F32 = jnp.float32


def _row3(x_ref, y, cin, wp):
    return x_ref[pl.ds(y, 3)].reshape(3 * cin, wp)


def _split_hi_lo(x):
    hi = x.astype(jnp.bfloat16)
    lo = (x - hi.astype(_F32)).astype(jnp.bfloat16)
    return hi, lo


def _stack_w(w3):
    hi = w3.astype(jnp.bfloat16)
    lo = (w3 - hi.astype(_F32)).astype(jnp.bfloat16)
    return jnp.concatenate([hi, lo], axis=-2)


def _dot3s(w2, b_hi, b_lo):
    cout = w2.shape[0] // 2
    s = jnp.dot(w2, b_hi, preferred_element_type=_F32)
    t = jnp.dot(w2[:cout], b_lo, preferred_element_type=_F32)
    return s[:cout] + s[cout:] + t


def _lane_mask(wp):
    lane = jax.lax.broadcasted_iota(jnp.int32, (1, wp), 1)
    return (lane >= 1) & (lane < wp - 1)


def _conv_row(wfun, x3, wp, pad_out, mask):
    x_hi, x_lo = _split_hi_lo(x3)
    p0 = _dot3s(wfun(0), x_hi, x_lo)
    p1 = _dot3s(wfun(1), x_hi, x_lo)
    p2 = _dot3s(wfun(2), x_hi, x_lo)
    cout = p0.shape[0]
    if pad_out:
        zc = jnp.zeros((cout, 1), _F32)
        s = (jnp.concatenate([zc, p0[:, :wp - 1]], axis=1) + p1
             + jnp.concatenate([p2[:, 1:], zc], axis=1))
        return s, mask
    return p0[:, 0:wp - 2] + p1[:, 1:wp - 1] + p2[:, 2:wp], None


def _pad_row(acc):
    cout = acc.shape[0]
    zcol = jnp.zeros((cout, 1), _F32)
    return jnp.concatenate([zcol, acc, zcol], axis=1)


_UNROLL = 8


def _conv_pass(src, dst, wfun, relu, pad_out, hh, ww, mask):
    cin = src.shape[1]
    cout = dst.shape[1]
    if pad_out:
        dst[0] = jnp.zeros((cout, ww + 2), _F32)
        dst[hh + 1] = jnp.zeros((cout, ww + 2), _F32)

    def rows(it, carry):
        y0 = it * _UNROLL
        for u in range(_UNROLL):
            y = y0 + u
            x3 = _row3(src, y, cin, ww + 2)
            s, m = _conv_row(wfun, x3, ww + 2, pad_out, mask)
            if relu:
                s = jnp.maximum(s, 0.0)
            if pad_out:
                dst[y + 1] = jnp.where(m, s, 0.0)
            else:
                dst[y] = s
        return carry

    jax.lax.fori_loop(0, hh // _UNROLL, rows, 0)


def _enc3_kernel(x_ref, w0_ref, w1_ref, w2_ref, o_ref, s1_ref, s2_ref,
                 *, hh, ww):
    mask = _lane_mask(ww + 2)
    _conv_pass(x_ref, s1_ref, lambda dx: w0_ref[dx], True, True, hh, ww, mask)
    _conv_pass(s1_ref, s2_ref, lambda dx: w1_ref[dx], True, True, hh, ww,
               mask)
    _conv_pass(s2_ref, o_ref, lambda dx: w2_ref[dx], True, True, hh, ww, mask)


def _enc3(xp, w0, w1, w2):
    hp, _, wp = xp.shape
    hh, ww = hp - 2, wp - 2
    c = w0.shape[1] // 2
    return pl.pallas_call(
        functools.partial(_enc3_kernel, hh=hh, ww=ww),
        out_shape=jax.ShapeDtypeStruct((hp, c, wp), _F32),
        scratch_shapes=[pltpu.VMEM((hp, c, wp), _F32),
                        pltpu.VMEM((hp, c, wp), _F32)],
    )(xp, w0, w1, w2)


def _fusion_kernel(v_ref, g_ref, i_ref, l_ref, ws_ref, wb0_ref, wb1_ref,
                   wb2_ref, wo_ref, o_ref, f_ref, s1_ref, s2_ref, *, hh, ww):
    cpad = f_ref.shape[1] - v_ref.shape[1]
    wp = ww + 2
    mask = _lane_mask(wp)

    def addrows(it, carry):
        y0 = it * 2
        for u in range(2):
            y = y0 + u
            s = v_ref[y] + g_ref[y] + (i_ref[y] + l_ref[y])
            f_ref[y] = jnp.concatenate([s, jnp.zeros((cpad, wp), _F32)],
                                       axis=0)
        return carry

    jax.lax.fori_loop(0, (hh + 2) // 2, addrows, 0)

    _conv_pass(f_ref, s1_ref, lambda dx: ws_ref[dx], True, True, hh, ww, mask)
    _conv_pass(s1_ref, s2_ref, lambda dx: wb0_ref[dx], True, True, hh, ww,
               mask)
    _conv_pass(s2_ref, s1_ref, lambda dx: wb1_ref[dx], True, True, hh, ww,
               mask)
    _conv_pass(s1_ref, s2_ref, lambda dx: wb2_ref[dx], True, True, hh, ww,
               mask)
    _conv_pass(s2_ref, o_ref, lambda dx: wo_ref[dx], False, False, hh, ww,
               mask)


def _fusion(vp, gp, ip, lp, ws, wb0, wb1, wb2, wo):
    hp, _, wp = vp.shape
    hh, ww = hp - 2, wp - 2
    c = ws.shape[1] // 2
    cout = wo.shape[1] // 2
    return pl.pallas_call(
        functools.partial(_fusion_kernel, hh=hh, ww=ww),
        out_shape=jax.ShapeDtypeStruct((hh, cout, ww), _F32),
        scratch_shapes=[pltpu.VMEM((hp, 8, wp), _F32),
                        pltpu.VMEM((hp, c, wp), _F32),
                        pltpu.VMEM((hp, c, wp), _F32)],
    )(vp, gp, ip, lp, ws, wb0, wb1, wb2, wo)


def _musig_kernel(h_ref, wms_ref, mu_ref, sg_ref, *, hh, ww):
    c = mu_ref.shape[1]

    def rows(it, carry):
        y0 = it * _UNROLL
        for u in range(_UNROLL):
            y = y0 + u
            hrow = h_ref[y + 1, :, 1:ww + 1]
            h_hi, h_lo = _split_hi_lo(hrow)
            ms = _dot3s(wms_ref[...], h_hi, h_lo)
            mu_ref[y] = ms[:c]
            raw = ms[c:]
            sp = jnp.maximum(raw, 0.0) + jnp.log1p(jnp.exp(-jnp.abs(raw)))
            sg_ref[y] = sp + 1e-6
        return carry

    jax.lax.fori_loop(0, hh // _UNROLL, rows, 0)


def _musig(hp, wms):
    hp_, c4 = hp.shape[0], wms.shape[0]
    hh, ww = hp_ - 2, hp.shape[2] - 2
    c = c4 // 4
    return pl.pallas_call(
        functools.partial(_musig_kernel, hh=hh, ww=ww),
        out_shape=(jax.ShapeDtypeStruct((hh, c, ww), _F32),
                   jax.ShapeDtypeStruct((hh, c, ww), _F32)),
    )(hp, wms)


def _viz_kernel(mu_ref, sg_ref, eps_ref, wr_ref, bm_ref, z_ref, rt_ref,
                acc_ref, *, hh, ww):
    c = mu_ref.shape[1]
    z_ref[0] = jnp.zeros((c, ww + 2), _F32)
    z_ref[hh + 1] = jnp.zeros((c, ww + 2), _F32)
    acc_ref[...] = jnp.zeros((c, ww), _F32)

    def rows(it, carry):
        y0 = it * _UNROLL
        acc = acc_ref[...]
        for u in range(_UNROLL):
            y = y0 + u
            z = mu_ref[y] + jnp.sqrt(sg_ref[y]) * eps_ref[y]
            z_ref[y + 1] = _pad_row(z)
            acc = acc + z
        acc_ref[...] = acc
        return carry

    jax.lax.fori_loop(0, hh // _UNROLL, rows, 0)

    pooled_mat = jnp.dot(wr_ref[...], acc_ref[...], precision=_PREC)
    logits = jnp.sum(pooled_mat, axis=1, keepdims=True) / (hh * ww)
    logits = logits + bm_ref[...]
    sub = jax.lax.broadcasted_iota(jnp.int32, (8, 1), 0)
    neg = jnp.float32(-1e30)
    m1 = jnp.max(logits)
    i1 = -jnp.max(jnp.where(logits == m1, -sub.astype(_F32), neg))
    masked = jnp.where(sub.astype(_F32) == i1, neg, logits)
    m2 = jnp.max(masked)
    i2 = -jnp.max(jnp.where(masked == m2, -sub.astype(_F32), neg))
    e1 = jnp.exp(m1 - m1)
    e2 = jnp.exp(m2 - m1)
    g1 = e1 / (e1 + e2)
    g2 = e2 / (e1 + e2)
    out = jnp.where(sub == 0, i1,
          jnp.where(sub == 1, i2,
          jnp.where(sub == 2, g1,
          jnp.where(sub == 3, g2, 0.0))))
    rt_ref[...] = jnp.broadcast_to(out, (8, 128))


def _viz(mu, sg, eps, wr, bmask):
    hh, c, ww = mu.shape
    return pl.pallas_call(
        functools.partial(_viz_kernel, hh=hh, ww=ww),
        out_shape=(jax.ShapeDtypeStruct((hh + 2, c, ww + 2), _F32),
                   jax.ShapeDtypeStruct((8, 128), _F32)),
        scratch_shapes=[pltpu.VMEM((c, ww), _F32)],
    )(mu, sg, eps, wr, bmask)


def _moe_kernel(z_ref, w1_ref, w2_ref, idx_ref, gate_ref, o_ref, a_ref,
                *, hh, ww):
    c = z_ref.shape[1]
    cout = o_ref.shape[1]
    o_ref[0] = jnp.zeros((cout, ww + 2), _F32)
    o_ref[hh + 1] = jnp.zeros((cout, ww + 2), _F32)
    a_ref[0] = jnp.zeros((c, ww + 2), _F32)
    a_ref[hh + 1] = jnp.zeros((c, ww + 2), _F32)
    mask = _lane_mask(ww + 2)

    for k in (0, 1):
        e = idx_ref[k]
        g = gate_ref[k]

        def rows1(it, carry):
            y0 = it * _UNROLL
            for u in range(_UNROLL):
                y = y0 + u
                x3 = _row3(z_ref, y, c, ww + 2)
                s, m = _conv_row(lambda dx: w1_ref[e, dx], x3, ww + 2, True,
                                 mask)
                a_ref[y + 1] = jnp.where(m, jnp.maximum(s, 0.0), 0.0)
            return carry

        jax.lax.fori_loop(0, hh // _UNROLL, rows1, 0)

        def rows2(it, carry):
            y0 = it * _UNROLL
            for u in range(_UNROLL):
                y = y0 + u
                x3 = _row3(a_ref, y, c, ww + 2)
                s, m = _conv_row(lambda dx: w2_ref[e, dx], x3, ww + 2, True,
                                 mask)
                s = jnp.where(m, s * g, 0.0)
                if k == 0:
                    o_ref[y + 1] = s
                else:
                    o_ref[y + 1] = o_ref[y + 1] + s
            return carry

        jax.lax.fori_loop(0, hh // _UNROLL, rows2, 0)


def _moe(zp, w1, w2, idx, gate):
    hp, c, wp = zp.shape
    hh, ww = hp - 2, wp - 2
    cout = w2.shape[2] // 2
    return pl.pallas_call(
        functools.partial(_moe_kernel, hh=hh, ww=ww),
        out_shape=jax.ShapeDtypeStruct((hp, cout, wp), _F32),
        in_specs=[pl.BlockSpec(memory_space=pltpu.VMEM),
                  pl.BlockSpec(memory_space=pltpu.VMEM),
                  pl.BlockSpec(memory_space=pltpu.VMEM),
                  pl.BlockSpec(memory_space=pltpu.SMEM),
                  pl.BlockSpec(memory_space=pltpu.SMEM)],
        scratch_shapes=[pltpu.VMEM((hp, c, wp), _F32)],
    )(zp, w1, w2, idx, gate)


def _prep_w3(w, cin_pad=None):
    cout, cin = w.shape[0], w.shape[1]
    if cin_pad is not None and cin_pad > cin:
        w = jnp.pad(w, ((0, 0), (0, cin_pad - cin), (0, 0), (0, 0)))
        cin = cin_pad
    return _stack_w(jnp.transpose(w, (3, 0, 2, 1)).reshape(3, cout, 3 * cin))


def _prep_wexp(w):
    e, cout, cin = w.shape[0], w.shape[1], w.shape[2]
    return _stack_w(
        jnp.transpose(w, (0, 4, 1, 3, 2)).reshape(e, 3, cout, 3 * cin))


def _to_hcw_pad(x, cpad=None):
    t = jnp.transpose(x[0], (1, 0, 2))
    extra = 0 if cpad is None else max(0, cpad - t.shape[1])
    return jnp.pad(t, ((1, 1), (0, extra), (1, 1)))


def _to_hcw(x):
    return jnp.transpose(x[0], (1, 0, 2))


def _from_hcw(x):
    return jnp.transpose(x, (1, 0, 2))[None]


def _unpad(xp):
    return xp[1:-1, :, 1:-1]


def _branch(x_nchw, eps_nchw, w_stem, w_blocks, w_mu, w_sig, w_router,
            b_router, w_exp1, w_exp2):
    xp = _to_hcw_pad(x_nchw, cpad=8)
    h = _enc3(xp, _prep_w3(w_stem, cin_pad=8), _prep_w3(w_blocks[0]),
              _prep_w3(w_blocks[1]))

    wms = _stack_w(jnp.concatenate([w_mu[:, :, 0, 0], w_sig[:, :, 0, 0]],
                                   axis=0))
    mu, sg2 = _musig(h, wms)

    e = w_router.shape[0]
    wr = jnp.pad(w_router, ((0, 8 - e), (0, 0)))
    bmask = jnp.pad(b_router, (0, 8 - e),
                    constant_values=-1e30).reshape(8, 1).astype(_F32)
    zp, rt = _viz(mu, sg2, _to_hcw(eps_nchw), wr, bmask)

    idx = rt[0:2, 0].astype(jnp.int32)
    gate = rt[2:4, 0]
    dec = _moe(zp, _prep_wexp(w_exp1), _prep_wexp(w_exp2), idx, gate)
    return dec, mu, sg2


def kernel(i, v, eps_i, eps_v, W_ie_stem, W_ie_blocks, W_i_mu, W_i_sig,
           W_i_router, b_i_router, W_i_exp1, W_i_exp2, W_ve_stem, W_ve_blocks,
           W_v_mu, W_v_sig, W_v_router, b_v_router, W_v_exp1, W_v_exp2,
           W_f_stem, W_f_blocks, W_f_out):
    lp, mu_l, sg_l = _branch(i, eps_i, W_ie_stem, W_ie_blocks, W_i_mu, W_i_sig,
                             W_i_router, b_i_router, W_i_exp1, W_i_exp2)
    gp, mu_g, sg_g = _branch(v, eps_v, W_ve_stem, W_ve_blocks, W_v_mu, W_v_sig,
                             W_v_router, b_v_router, W_v_exp1, W_v_exp2)

    fusion = _fusion(_to_hcw_pad(v), gp, _to_hcw_pad(i), lp,
                     _prep_w3(W_f_stem, cin_pad=8), _prep_w3(W_f_blocks[0]),
                     _prep_w3(W_f_blocks[1]), _prep_w3(W_f_blocks[2]),
                     _prep_w3(W_f_out))

    zz = jnp.zeros((1, 64, 224, 224), _F32)
    return (_from_hcw(fusion), _from_hcw(_unpad(lp)), _from_hcw(_unpad(gp)),
            zz + mu_l[0, 0, 0], zz + sg_l[0, 0, 0], zz + mu_g[0, 0, 0],
            zz + sg_g[0, 0, 0])


# speedup vs baseline: 1.0135x; 1.0135x over previous
import functools

import jax
import jax.numpy as jnp
from jax.experimental import pallas as pl
from jax.experimental.pallas import tpu as pltpu

_PREC = jax.lax.Precision.HIGHEST
_CPREC = jax.lax.Precision.DEFAULT
_F32 = jnp.float32


def _row3(x_ref, y, cin, wp):
    return x_ref[pl.ds(y, 3)].reshape(3 * cin, wp)


def _split_hi_lo(x):
    hi = x.astype(jnp.bfloat16)
    lo = (x - hi.astype(_F32)).astype(jnp.bfloat16)
    return hi, lo


def _stack_w(w3):
    hi = w3.astype(jnp.bfloat16)
    lo = (w3 - hi.astype(_F32)).astype(jnp.bfloat16)
    return jnp.concatenate([hi, lo], axis=-2)


def _dot3s(w2, b_hi, b_lo):
    cout = w2.shape[0] // 2
    s = jnp.dot(w2, b_hi, preferred_element_type=_F32)
    t = jnp.dot(w2[:cout], b_lo, preferred_element_type=_F32)
    return s[:cout] + s[cout:] + t


def _lane_mask(wp):
    lane = jax.lax.broadcasted_iota(jnp.int32, (1, wp), 1)
    return (lane >= 1) & (lane < wp - 1)


def _conv_row(wfun, x_hi, x_lo, wp, pad_out, mask):
    p0 = _dot3s(wfun(0), x_hi, x_lo)
    p1 = _dot3s(wfun(1), x_hi, x_lo)
    p2 = _dot3s(wfun(2), x_hi, x_lo)
    cout = p0.shape[0]
    if pad_out:
        zc = jnp.zeros((cout, 1), _F32)
        s = (jnp.concatenate([zc, p0[:, :wp - 1]], axis=1) + p1
             + jnp.concatenate([p2[:, 1:], zc], axis=1))
        return s, mask
    return p0[:, 0:wp - 2] + p1[:, 1:wp - 1] + p2[:, 2:wp], None


def _pad_row(acc):
    cout = acc.shape[0]
    zcol = jnp.zeros((cout, 1), _F32)
    return jnp.concatenate([zcol, acc, zcol], axis=1)


_UNROLL = 8


def _load3_hilo(src, y, wp):
    if isinstance(src, tuple):
        cin = src[0].shape[1]
        return (_row3(src[0], y, cin, wp), _row3(src[1], y, cin, wp))
    return _split_hi_lo(_row3(src, y, src.shape[1], wp))


def _store_row(dst, y, s):
    if isinstance(dst, tuple):
        sh, sl = _split_hi_lo(s)
        dst[0][y] = sh
        dst[1][y] = sl
    else:
        dst[y] = s


def _zero_row(dst, y, cout, wp):
    if isinstance(dst, tuple):
        dst[0][y] = jnp.zeros((cout, wp), jnp.bfloat16)
        dst[1][y] = jnp.zeros((cout, wp), jnp.bfloat16)
    else:
        dst[y] = jnp.zeros((cout, wp), _F32)


def _conv_pass(src, dst, wfun, relu, pad_out, hh, ww, mask):
    cout = (dst[0] if isinstance(dst, tuple) else dst).shape[1]
    if pad_out:
        _zero_row(dst, 0, cout, ww + 2)
        _zero_row(dst, hh + 1, cout, ww + 2)

    def rows(it, carry):
        y0 = it * _UNROLL
        for u in range(_UNROLL):
            y = y0 + u
            x3h, x3l = _load3_hilo(src, y, ww + 2)
            s, m = _conv_row(wfun, x3h, x3l, ww + 2, pad_out, mask)
            if relu:
                s = jnp.maximum(s, 0.0)
            if pad_out:
                _store_row(dst, y + 1, jnp.where(m, s, 0.0))
            else:
                _store_row(dst, y, s)
        return carry

    jax.lax.fori_loop(0, hh // _UNROLL, rows, 0)


def _enc3_kernel(x_ref, w0_ref, w1_ref, w2_ref, o_ref, s1h, s1l, s2h, s2l,
                 *, hh, ww):
    mask = _lane_mask(ww + 2)
    _conv_pass(x_ref, (s1h, s1l), lambda dx: w0_ref[dx], True, True, hh, ww,
               mask)
    _conv_pass((s1h, s1l), (s2h, s2l), lambda dx: w1_ref[dx], True, True, hh,
               ww, mask)
    _conv_pass((s2h, s2l), o_ref, lambda dx: w2_ref[dx], True, True, hh, ww,
               mask)


def _enc3(xp, w0, w1, w2):
    hp, _, wp = xp.shape
    hh, ww = hp - 2, wp - 2
    c = w0.shape[1] // 2
    bf = jnp.bfloat16
    return pl.pallas_call(
        functools.partial(_enc3_kernel, hh=hh, ww=ww),
        out_shape=jax.ShapeDtypeStruct((hp, c, wp), _F32),
        scratch_shapes=[pltpu.VMEM((hp, c, wp), bf),
                        pltpu.VMEM((hp, c, wp), bf),
                        pltpu.VMEM((hp, c, wp), bf),
                        pltpu.VMEM((hp, c, wp), bf)],
    )(xp, w0, w1, w2)


def _fusion_kernel(v_ref, g_ref, i_ref, l_ref, ws_ref, wb0_ref, wb1_ref,
                   wb2_ref, wo_ref, o_ref, f_ref, s1h, s1l, s2h, s2l,
                   *, hh, ww):
    cpad = f_ref.shape[1] - v_ref.shape[1]
    wp = ww + 2
    mask = _lane_mask(wp)

    def addrows(it, carry):
        y0 = it * 2
        for u in range(2):
            y = y0 + u
            s = v_ref[y] + g_ref[y] + (i_ref[y] + l_ref[y])
            f_ref[y] = jnp.concatenate([s, jnp.zeros((cpad, wp), _F32)],
                                       axis=0)
        return carry

    jax.lax.fori_loop(0, (hh + 2) // 2, addrows, 0)

    s1 = (s1h, s1l)
    s2 = (s2h, s2l)
    _conv_pass(f_ref, s1, lambda dx: ws_ref[dx], True, True, hh, ww, mask)
    _conv_pass(s1, s2, lambda dx: wb0_ref[dx], True, True, hh, ww, mask)
    _conv_pass(s2, s1, lambda dx: wb1_ref[dx], True, True, hh, ww, mask)
    _conv_pass(s1, s2, lambda dx: wb2_ref[dx], True, True, hh, ww, mask)
    _conv_pass(s2, o_ref, lambda dx: wo_ref[dx], False, False, hh, ww, mask)


def _fusion(vp, gp, ip, lp, ws, wb0, wb1, wb2, wo):
    hp, _, wp = vp.shape
    hh, ww = hp - 2, wp - 2
    c = ws.shape[1] // 2
    cout = wo.shape[1] // 2
    bf = jnp.bfloat16
    return pl.pallas_call(
        functools.partial(_fusion_kernel, hh=hh, ww=ww),
        out_shape=jax.ShapeDtypeStruct((hh, cout, ww), _F32),
        scratch_shapes=[pltpu.VMEM((hp, 8, wp), _F32),
                        pltpu.VMEM((hp, c, wp), bf),
                        pltpu.VMEM((hp, c, wp), bf),
                        pltpu.VMEM((hp, c, wp), bf),
                        pltpu.VMEM((hp, c, wp), bf)],
    )(vp, gp, ip, lp, ws, wb0, wb1, wb2, wo)


def _musig_kernel(h_ref, wms_ref, mu_ref, sg_ref, *, hh, ww):
    c = mu_ref.shape[1]

    def rows(it, carry):
        y0 = it * _UNROLL
        for u in range(_UNROLL):
            y = y0 + u
            hrow = h_ref[y + 1, :, 1:ww + 1]
            h_hi, h_lo = _split_hi_lo(hrow)
            ms = _dot3s(wms_ref[...], h_hi, h_lo)
            mu_ref[y] = ms[:c]
            raw = ms[c:]
            sp = jnp.maximum(raw, 0.0) + jnp.log1p(jnp.exp(-jnp.abs(raw)))
            sg_ref[y] = sp + 1e-6
        return carry

    jax.lax.fori_loop(0, hh // _UNROLL, rows, 0)


def _musig(hp, wms):
    hp_, c4 = hp.shape[0], wms.shape[0]
    hh, ww = hp_ - 2, hp.shape[2] - 2
    c = c4 // 4
    return pl.pallas_call(
        functools.partial(_musig_kernel, hh=hh, ww=ww),
        out_shape=(jax.ShapeDtypeStruct((hh, c, ww), _F32),
                   jax.ShapeDtypeStruct((hh, c, ww), _F32)),
    )(hp, wms)


def _viz_kernel(mu_ref, sg_ref, eps_ref, wr_ref, bm_ref, zh_ref, zl_ref,
                rt_ref, acc_ref, *, hh, ww):
    c = mu_ref.shape[1]
    _zero_row((zh_ref, zl_ref), 0, c, ww + 2)
    _zero_row((zh_ref, zl_ref), hh + 1, c, ww + 2)
    acc_ref[...] = jnp.zeros((c, ww), _F32)

    def rows(it, carry):
        y0 = it * _UNROLL
        acc = acc_ref[...]
        for u in range(_UNROLL):
            y = y0 + u
            z = mu_ref[y] + jnp.sqrt(sg_ref[y]) * eps_ref[y]
            _store_row((zh_ref, zl_ref), y + 1, _pad_row(z))
            acc = acc + z
        acc_ref[...] = acc
        return carry

    jax.lax.fori_loop(0, hh // _UNROLL, rows, 0)

    pooled_mat = jnp.dot(wr_ref[...], acc_ref[...], precision=_PREC)
    logits = jnp.sum(pooled_mat, axis=1, keepdims=True) / (hh * ww)
    logits = logits + bm_ref[...]
    sub = jax.lax.broadcasted_iota(jnp.int32, (8, 1), 0)
    neg = jnp.float32(-1e30)
    m1 = jnp.max(logits)
    i1 = -jnp.max(jnp.where(logits == m1, -sub.astype(_F32), neg))
    masked = jnp.where(sub.astype(_F32) == i1, neg, logits)
    m2 = jnp.max(masked)
    i2 = -jnp.max(jnp.where(masked == m2, -sub.astype(_F32), neg))
    e1 = jnp.exp(m1 - m1)
    e2 = jnp.exp(m2 - m1)
    g1 = e1 / (e1 + e2)
    g2 = e2 / (e1 + e2)
    out = jnp.where(sub == 0, i1,
          jnp.where(sub == 1, i2,
          jnp.where(sub == 2, g1,
          jnp.where(sub == 3, g2, 0.0))))
    rt_ref[...] = jnp.broadcast_to(out, (8, 128))


def _viz(mu, sg, eps, wr, bmask):
    hh, c, ww = mu.shape
    bf = jnp.bfloat16
    return pl.pallas_call(
        functools.partial(_viz_kernel, hh=hh, ww=ww),
        out_shape=(jax.ShapeDtypeStruct((hh + 2, c, ww + 2), bf),
                   jax.ShapeDtypeStruct((hh + 2, c, ww + 2), bf),
                   jax.ShapeDtypeStruct((8, 128), _F32)),
        scratch_shapes=[pltpu.VMEM((c, ww), _F32)],
    )(mu, sg, eps, wr, bmask)


def _moe_kernel(zh_ref, zl_ref, w1_ref, w2_ref, idx_ref, gate_ref, o_ref,
                ah_ref, al_ref, *, hh, ww):
    c = zh_ref.shape[1]
    cout = o_ref.shape[1]
    o_ref[0] = jnp.zeros((cout, ww + 2), _F32)
    o_ref[hh + 1] = jnp.zeros((cout, ww + 2), _F32)
    a = (ah_ref, al_ref)
    _zero_row(a, 0, c, ww + 2)
    _zero_row(a, hh + 1, c, ww + 2)
    mask = _lane_mask(ww + 2)

    for k in (0, 1):
        e = idx_ref[k]
        g = gate_ref[k]

        def rows1(it, carry):
            y0 = it * _UNROLL
            for u in range(_UNROLL):
                y = y0 + u
                x3h, x3l = _load3_hilo((zh_ref, zl_ref), y, ww + 2)
                s, m = _conv_row(lambda dx: w1_ref[e, dx], x3h, x3l, ww + 2,
                                 True, mask)
                _store_row(a, y + 1,
                           jnp.where(m, jnp.maximum(s, 0.0), 0.0))
            return carry

        jax.lax.fori_loop(0, hh // _UNROLL, rows1, 0)

        def rows2(it, carry):
            y0 = it * _UNROLL
            for u in range(_UNROLL):
                y = y0 + u
                x3h, x3l = _load3_hilo(a, y, ww + 2)
                s, m = _conv_row(lambda dx: w2_ref[e, dx], x3h, x3l, ww + 2,
                                 True, mask)
                s = jnp.where(m, s * g, 0.0)
                if k == 0:
                    o_ref[y + 1] = s
                else:
                    o_ref[y + 1] = o_ref[y + 1] + s
            return carry

        jax.lax.fori_loop(0, hh // _UNROLL, rows2, 0)


def _moe(zh, zl, w1, w2, idx, gate):
    hp, c, wp = zh.shape
    hh, ww = hp - 2, wp - 2
    cout = w2.shape[2] // 2
    bf = jnp.bfloat16
    return pl.pallas_call(
        functools.partial(_moe_kernel, hh=hh, ww=ww),
        out_shape=jax.ShapeDtypeStruct((hp, cout, wp), _F32),
        in_specs=[pl.BlockSpec(memory_space=pltpu.VMEM),
                  pl.BlockSpec(memory_space=pltpu.VMEM),
                  pl.BlockSpec(memory_space=pltpu.VMEM),
                  pl.BlockSpec(memory_space=pltpu.VMEM),
                  pl.BlockSpec(memory_space=pltpu.SMEM),
                  pl.BlockSpec(memory_space=pltpu.SMEM)],
        scratch_shapes=[pltpu.VMEM((hp, c, wp), bf),
                        pltpu.VMEM((hp, c, wp), bf)],
    )(zh, zl, w1, w2, idx, gate)


def _prep_w3(w, cin_pad=None):
    cout, cin = w.shape[0], w.shape[1]
    if cin_pad is not None and cin_pad > cin:
        w = jnp.pad(w, ((0, 0), (0, cin_pad - cin), (0, 0), (0, 0)))
        cin = cin_pad
    return _stack_w(jnp.transpose(w, (3, 0, 2, 1)).reshape(3, cout, 3 * cin))


def _prep_wexp(w):
    e, cout, cin = w.shape[0], w.shape[1], w.shape[2]
    return _stack_w(
        jnp.transpose(w, (0, 4, 1, 3, 2)).reshape(e, 3, cout, 3 * cin))


def _to_hcw_pad(x, cpad=None):
    t = jnp.transpose(x[0], (1, 0, 2))
    extra = 0 if cpad is None else max(0, cpad - t.shape[1])
    return jnp.pad(t, ((1, 1), (0, extra), (1, 1)))


def _to_hcw(x):
    return jnp.transpose(x[0], (1, 0, 2))


def _from_hcw(x):
    return jnp.transpose(x, (1, 0, 2))[None]


def _unpad(xp):
    return xp[1:-1, :, 1:-1]


def _branch(x_nchw, eps_nchw, w_stem, w_blocks, w_mu, w_sig, w_router,
            b_router, w_exp1, w_exp2):
    xp = _to_hcw_pad(x_nchw, cpad=8)
    h = _enc3(xp, _prep_w3(w_stem, cin_pad=8), _prep_w3(w_blocks[0]),
              _prep_w3(w_blocks[1]))

    wms = _stack_w(jnp.concatenate([w_mu[:, :, 0, 0], w_sig[:, :, 0, 0]],
                                   axis=0))
    mu, sg2 = _musig(h, wms)

    e = w_router.shape[0]
    wr = jnp.pad(w_router, ((0, 8 - e), (0, 0)))
    bmask = jnp.pad(b_router, (0, 8 - e),
                    constant_values=-1e30).reshape(8, 1).astype(_F32)
    zh, zl, rt = _viz(mu, sg2, _to_hcw(eps_nchw), wr, bmask)

    idx = rt[0:2, 0].astype(jnp.int32)
    gate = rt[2:4, 0]
    dec = _moe(zh, zl, _prep_wexp(w_exp1), _prep_wexp(w_exp2), idx, gate)
    return dec, mu, sg2


def kernel(i, v, eps_i, eps_v, W_ie_stem, W_ie_blocks, W_i_mu, W_i_sig,
           W_i_router, b_i_router, W_i_exp1, W_i_exp2, W_ve_stem, W_ve_blocks,
           W_v_mu, W_v_sig, W_v_router, b_v_router, W_v_exp1, W_v_exp2,
           W_f_stem, W_f_blocks, W_f_out):
    lp, mu_l, sg_l = _branch(i, eps_i, W_ie_stem, W_ie_blocks, W_i_mu, W_i_sig,
                             W_i_router, b_i_router, W_i_exp1, W_i_exp2)
    gp, mu_g, sg_g = _branch(v, eps_v, W_ve_stem, W_ve_blocks, W_v_mu, W_v_sig,
                             W_v_router, b_v_router, W_v_exp1, W_v_exp2)

    fusion = _fusion(_to_hcw_pad(v), gp, _to_hcw_pad(i), lp,
                     _prep_w3(W_f_stem, cin_pad=8), _prep_w3(W_f_blocks[0]),
                     _prep_w3(W_f_blocks[1]), _prep_w3(W_f_blocks[2]),
                     _prep_w3(W_f_out))

    return (_from_hcw(fusion), _from_hcw(_unpad(lp)), _from_hcw(_unpad(gp)),
            _from_hcw(mu_l), _from_hcw(sg_l), _from_hcw(mu_g), _from_hcw(sg_g))


# speedup vs baseline: 1.1760x; 1.1603x over previous
import functools

import jax
import jax.numpy as jnp
from jax.experimental import pallas as pl
from jax.experimental.pallas import tpu as pltpu

_PREC = jax.lax.Precision.HIGHEST
_CPREC = jax.lax.Precision.DEFAULT
_F32 = jnp.float32


def _row3(x_ref, y, cin, wp):
    return x_ref[pl.ds(y, 3)].reshape(3 * cin, wp)


def _split_hi_lo(x):
    hi = x.astype(jnp.bfloat16)
    lo = (x - hi.astype(_F32)).astype(jnp.bfloat16)
    return hi, lo


def _stack_w(w3):
    hi = w3.astype(jnp.bfloat16)
    lo = (w3 - hi.astype(_F32)).astype(jnp.bfloat16)
    return jnp.concatenate([hi, lo], axis=-2)


def _dot3s(w2, b_hi, b_lo):
    cout = w2.shape[0] // 2
    s = jnp.dot(w2, b_hi, preferred_element_type=_F32)
    t = jnp.dot(w2[:cout], b_lo, preferred_element_type=_F32)
    return s[:cout] + s[cout:] + t


def _dot3_taps(wall, whi3, b_hi, b_lo):
    co = whi3.shape[0] // 3
    s = jnp.dot(wall, b_hi, preferred_element_type=_F32)
    t = jnp.dot(whi3, b_lo, preferred_element_type=_F32)

    def p(dx):
        return (s[2 * co * dx:2 * co * dx + co]
                + s[2 * co * dx + co:2 * co * (dx + 1)]
                + t[co * dx:co * (dx + 1)])

    return p(0), p(1), p(2)


def _lane_mask(wp):
    lane = jax.lax.broadcasted_iota(jnp.int32, (1, wp), 1)
    return (lane >= 1) & (lane < wp - 1)


def _conv_row(wfun, x_hi, x_lo, wp, pad_out, mask):
    wall, whi3 = wfun()
    p0, p1, p2 = _dot3_taps(wall, whi3, x_hi, x_lo)
    cout = p0.shape[0]
    if pad_out:
        zc = jnp.zeros((cout, 1), _F32)
        s = (jnp.concatenate([zc, p0[:, :wp - 1]], axis=1) + p1
             + jnp.concatenate([p2[:, 1:], zc], axis=1))
        return s, mask
    return p0[:, 0:wp - 2] + p1[:, 1:wp - 1] + p2[:, 2:wp], None


def _pad_row(acc):
    cout = acc.shape[0]
    zcol = jnp.zeros((cout, 1), _F32)
    return jnp.concatenate([zcol, acc, zcol], axis=1)


_UNROLL = 8


def _load3_hilo(src, y, wp):
    if isinstance(src, tuple):
        cin = src[0].shape[1]
        return (_row3(src[0], y, cin, wp), _row3(src[1], y, cin, wp))
    return _split_hi_lo(_row3(src, y, src.shape[1], wp))


def _store_row(dst, y, s):
    if isinstance(dst, tuple):
        sh, sl = _split_hi_lo(s)
        dst[0][y] = sh
        dst[1][y] = sl
    else:
        dst[y] = s


def _zero_row(dst, y, cout, wp):
    if isinstance(dst, tuple):
        dst[0][y] = jnp.zeros((cout, wp), jnp.bfloat16)
        dst[1][y] = jnp.zeros((cout, wp), jnp.bfloat16)
    else:
        dst[y] = jnp.zeros((cout, wp), _F32)


def _conv_pass(src, dst, wfun, relu, pad_out, hh, ww, mask):
    cout = (dst[0] if isinstance(dst, tuple) else dst).shape[1]
    if pad_out:
        _zero_row(dst, 0, cout, ww + 2)
        _zero_row(dst, hh + 1, cout, ww + 2)

    def rows(it, carry):
        y0 = it * _UNROLL
        for u in range(_UNROLL):
            y = y0 + u
            x3h, x3l = _load3_hilo(src, y, ww + 2)
            s, m = _conv_row(wfun, x3h, x3l, ww + 2, pad_out, mask)
            if relu:
                s = jnp.maximum(s, 0.0)
            if pad_out:
                _store_row(dst, y + 1, jnp.where(m, s, 0.0))
            else:
                _store_row(dst, y, s)
        return carry

    jax.lax.fori_loop(0, hh // _UNROLL, rows, 0)


def _enc3_kernel(x_ref, w0a, w0h, w1a, w1h, w2a, w2h, o_ref, s1h, s1l, s2h,
                 s2l, *, hh, ww):
    mask = _lane_mask(ww + 2)
    _conv_pass(x_ref, (s1h, s1l), lambda: (w0a[...], w0h[...]), True, True,
               hh, ww, mask)
    _conv_pass((s1h, s1l), (s2h, s2l), lambda: (w1a[...], w1h[...]), True,
               True, hh, ww, mask)
    _conv_pass((s2h, s2l), o_ref, lambda: (w2a[...], w2h[...]), True, True,
               hh, ww, mask)


def _enc3(xp, w0, w1, w2):
    hp, _, wp = xp.shape
    hh, ww = hp - 2, wp - 2
    c = w0[1].shape[0] // 3
    bf = jnp.bfloat16
    return pl.pallas_call(
        functools.partial(_enc3_kernel, hh=hh, ww=ww),
        out_shape=jax.ShapeDtypeStruct((hp, c, wp), _F32),
        scratch_shapes=[pltpu.VMEM((hp, c, wp), bf),
                        pltpu.VMEM((hp, c, wp), bf),
                        pltpu.VMEM((hp, c, wp), bf),
                        pltpu.VMEM((hp, c, wp), bf)],
    )(xp, w0[0], w0[1], w1[0], w1[1], w2[0], w2[1])


def _fusion_kernel(v_ref, g_ref, i_ref, l_ref, wsa, wsh, wb0a, wb0h, wb1a,
                   wb1h, wb2a, wb2h, woa, woh, o_ref, f_ref, s1h, s1l, s2h,
                   s2l, *, hh, ww):
    cpad = f_ref.shape[1] - v_ref.shape[1]
    wp = ww + 2
    mask = _lane_mask(wp)

    def addrows(it, carry):
        y0 = it * 2
        for u in range(2):
            y = y0 + u
            s = v_ref[y] + g_ref[y] + (i_ref[y] + l_ref[y])
            f_ref[y] = jnp.concatenate([s, jnp.zeros((cpad, wp), _F32)],
                                       axis=0)
        return carry

    jax.lax.fori_loop(0, (hh + 2) // 2, addrows, 0)

    s1 = (s1h, s1l)
    s2 = (s2h, s2l)
    _conv_pass(f_ref, s1, lambda: (wsa[...], wsh[...]), True, True, hh, ww,
               mask)
    _conv_pass(s1, s2, lambda: (wb0a[...], wb0h[...]), True, True, hh, ww,
               mask)
    _conv_pass(s2, s1, lambda: (wb1a[...], wb1h[...]), True, True, hh, ww,
               mask)
    _conv_pass(s1, s2, lambda: (wb2a[...], wb2h[...]), True, True, hh, ww,
               mask)
    _conv_pass(s2, o_ref, lambda: (woa[...], woh[...]), False, False, hh, ww,
               mask)


def _fusion(vp, gp, ip, lp, ws, wb0, wb1, wb2, wo):
    hp, _, wp = vp.shape
    hh, ww = hp - 2, wp - 2
    c = ws[1].shape[0] // 3
    cout = wo[1].shape[0] // 3
    bf = jnp.bfloat16
    return pl.pallas_call(
        functools.partial(_fusion_kernel, hh=hh, ww=ww),
        out_shape=jax.ShapeDtypeStruct((hh, cout, ww), _F32),
        scratch_shapes=[pltpu.VMEM((hp, 8, wp), _F32),
                        pltpu.VMEM((hp, c, wp), bf),
                        pltpu.VMEM((hp, c, wp), bf),
                        pltpu.VMEM((hp, c, wp), bf),
                        pltpu.VMEM((hp, c, wp), bf)],
    )(vp, gp, ip, lp, ws[0], ws[1], wb0[0], wb0[1], wb1[0], wb1[1], wb2[0],
      wb2[1], wo[0], wo[1])


def _musig_kernel(h_ref, wms_ref, mu_ref, sg_ref, *, hh, ww):
    c = mu_ref.shape[1]

    def rows(it, carry):
        y0 = it * _UNROLL
        for u in range(_UNROLL):
            y = y0 + u
            hrow = h_ref[y + 1, :, 1:ww + 1]
            h_hi, h_lo = _split_hi_lo(hrow)
            ms = _dot3s(wms_ref[...], h_hi, h_lo)
            mu_ref[y] = ms[:c]
            raw = ms[c:]
            sp = jnp.maximum(raw, 0.0) + jnp.log1p(jnp.exp(-jnp.abs(raw)))
            sg_ref[y] = sp + 1e-6
        return carry

    jax.lax.fori_loop(0, hh // _UNROLL, rows, 0)


def _musig(hp, wms):
    hp_, c4 = hp.shape[0], wms.shape[0]
    hh, ww = hp_ - 2, hp.shape[2] - 2
    c = c4 // 4
    return pl.pallas_call(
        functools.partial(_musig_kernel, hh=hh, ww=ww),
        out_shape=(jax.ShapeDtypeStruct((hh, c, ww), _F32),
                   jax.ShapeDtypeStruct((hh, c, ww), _F32)),
    )(hp, wms)


def _viz_kernel(mu_ref, sg_ref, eps_ref, wr_ref, bm_ref, zh_ref, zl_ref,
                rt_ref, acc_ref, *, hh, ww):
    c = mu_ref.shape[1]
    _zero_row((zh_ref, zl_ref), 0, c, ww + 2)
    _zero_row((zh_ref, zl_ref), hh + 1, c, ww + 2)
    acc_ref[...] = jnp.zeros((c, ww), _F32)

    def rows(it, carry):
        y0 = it * _UNROLL
        acc = acc_ref[...]
        for u in range(_UNROLL):
            y = y0 + u
            z = mu_ref[y] + jnp.sqrt(sg_ref[y]) * eps_ref[y]
            _store_row((zh_ref, zl_ref), y + 1, _pad_row(z))
            acc = acc + z
        acc_ref[...] = acc
        return carry

    jax.lax.fori_loop(0, hh // _UNROLL, rows, 0)

    pooled_mat = jnp.dot(wr_ref[...], acc_ref[...], precision=_PREC)
    logits = jnp.sum(pooled_mat, axis=1, keepdims=True) / (hh * ww)
    logits = logits + bm_ref[...]
    sub = jax.lax.broadcasted_iota(jnp.int32, (8, 1), 0)
    neg = jnp.float32(-1e30)
    m1 = jnp.max(logits)
    i1 = -jnp.max(jnp.where(logits == m1, -sub.astype(_F32), neg))
    masked = jnp.where(sub.astype(_F32) == i1, neg, logits)
    m2 = jnp.max(masked)
    i2 = -jnp.max(jnp.where(masked == m2, -sub.astype(_F32), neg))
    e1 = jnp.exp(m1 - m1)
    e2 = jnp.exp(m2 - m1)
    g1 = e1 / (e1 + e2)
    g2 = e2 / (e1 + e2)
    out = jnp.where(sub == 0, i1,
          jnp.where(sub == 1, i2,
          jnp.where(sub == 2, g1,
          jnp.where(sub == 3, g2, 0.0))))
    rt_ref[...] = jnp.broadcast_to(out, (8, 128))


def _viz(mu, sg, eps, wr, bmask):
    hh, c, ww = mu.shape
    bf = jnp.bfloat16
    return pl.pallas_call(
        functools.partial(_viz_kernel, hh=hh, ww=ww),
        out_shape=(jax.ShapeDtypeStruct((hh + 2, c, ww + 2), bf),
                   jax.ShapeDtypeStruct((hh + 2, c, ww + 2), bf),
                   jax.ShapeDtypeStruct((8, 128), _F32)),
        scratch_shapes=[pltpu.VMEM((c, ww), _F32)],
    )(mu, sg, eps, wr, bmask)


def _moe_kernel(zh_ref, zl_ref, w1a_ref, w1h_ref, w2a_ref, w2h_ref, idx_ref,
                gate_ref, o_ref, ah_ref, al_ref, *, hh, ww):
    c = zh_ref.shape[1]
    cout = o_ref.shape[1]
    o_ref[0] = jnp.zeros((cout, ww + 2), _F32)
    o_ref[hh + 1] = jnp.zeros((cout, ww + 2), _F32)
    a = (ah_ref, al_ref)
    _zero_row(a, 0, c, ww + 2)
    _zero_row(a, hh + 1, c, ww + 2)
    mask = _lane_mask(ww + 2)

    for k in (0, 1):
        e = idx_ref[k]
        g = gate_ref[k]

        def rows1(it, carry):
            y0 = it * _UNROLL
            for u in range(_UNROLL):
                y = y0 + u
                x3h, x3l = _load3_hilo((zh_ref, zl_ref), y, ww + 2)
                s, m = _conv_row(lambda: (w1a_ref[e], w1h_ref[e]), x3h, x3l,
                                 ww + 2, True, mask)
                _store_row(a, y + 1,
                           jnp.where(m, jnp.maximum(s, 0.0), 0.0))
            return carry

        jax.lax.fori_loop(0, hh // _UNROLL, rows1, 0)

        def rows2(it, carry):
            y0 = it * _UNROLL
            for u in range(_UNROLL):
                y = y0 + u
                x3h, x3l = _load3_hilo(a, y, ww + 2)
                s, m = _conv_row(lambda: (w2a_ref[e], w2h_ref[e]), x3h, x3l,
                                 ww + 2, True, mask)
                s = jnp.where(m, s * g, 0.0)
                if k == 0:
                    o_ref[y + 1] = s
                else:
                    o_ref[y + 1] = o_ref[y + 1] + s
            return carry

        jax.lax.fori_loop(0, hh // _UNROLL, rows2, 0)


def _moe(zh, zl, w1, w2, idx, gate):
    hp, c, wp = zh.shape
    hh, ww = hp - 2, wp - 2
    cout = w2[1].shape[1] // 3
    bf = jnp.bfloat16
    vm = pl.BlockSpec(memory_space=pltpu.VMEM)
    sm = pl.BlockSpec(memory_space=pltpu.SMEM)
    return pl.pallas_call(
        functools.partial(_moe_kernel, hh=hh, ww=ww),
        out_shape=jax.ShapeDtypeStruct((hp, cout, wp), _F32),
        in_specs=[vm, vm, vm, vm, vm, vm, sm, sm],
        scratch_shapes=[pltpu.VMEM((hp, c, wp), bf),
                        pltpu.VMEM((hp, c, wp), bf)],
    )(zh, zl, w1[0], w1[1], w2[0], w2[1], idx, gate)


def _prep_w3(w, cin_pad=None):
    cout, cin = w.shape[0], w.shape[1]
    if cin_pad is not None and cin_pad > cin:
        w = jnp.pad(w, ((0, 0), (0, cin_pad - cin), (0, 0), (0, 0)))
        cin = cin_pad
    w3s = _stack_w(jnp.transpose(w, (3, 0, 2, 1)).reshape(3, cout, 3 * cin))
    wall = w3s.reshape(6 * cout, 3 * cin)
    whi3 = w3s[:, :cout, :].reshape(3 * cout, 3 * cin)
    return wall, whi3


def _prep_wexp(w):
    e, cout, cin = w.shape[0], w.shape[1], w.shape[2]
    w3s = _stack_w(
        jnp.transpose(w, (0, 4, 1, 3, 2)).reshape(e, 3, cout, 3 * cin))
    wall = w3s.reshape(e, 6 * cout, 3 * cin)
    whi3 = w3s[:, :, :cout, :].reshape(e, 3 * cout, 3 * cin)
    return wall, whi3


def _to_hcw_pad(x, cpad=None):
    t = jnp.transpose(x[0], (1, 0, 2))
    extra = 0 if cpad is None else max(0, cpad - t.shape[1])
    return jnp.pad(t, ((1, 1), (0, extra), (1, 1)))


def _to_hcw(x):
    return jnp.transpose(x[0], (1, 0, 2))


def _from_hcw(x):
    return jnp.transpose(x, (1, 0, 2))[None]


def _unpad(xp):
    return xp[1:-1, :, 1:-1]


def _branch(x_nchw, eps_nchw, w_stem, w_blocks, w_mu, w_sig, w_router,
            b_router, w_exp1, w_exp2):
    xp = _to_hcw_pad(x_nchw, cpad=8)
    h = _enc3(xp, _prep_w3(w_stem, cin_pad=8), _prep_w3(w_blocks[0]),
              _prep_w3(w_blocks[1]))

    wms = _stack_w(jnp.concatenate([w_mu[:, :, 0, 0], w_sig[:, :, 0, 0]],
                                   axis=0))
    mu, sg2 = _musig(h, wms)

    e = w_router.shape[0]
    wr = jnp.pad(w_router, ((0, 8 - e), (0, 0)))
    bmask = jnp.pad(b_router, (0, 8 - e),
                    constant_values=-1e30).reshape(8, 1).astype(_F32)
    zh, zl, rt = _viz(mu, sg2, _to_hcw(eps_nchw), wr, bmask)

    idx = rt[0:2, 0].astype(jnp.int32)
    gate = rt[2:4, 0]
    dec = _moe(zh, zl, _prep_wexp(w_exp1), _prep_wexp(w_exp2), idx, gate)
    return dec, mu, sg2


def kernel(i, v, eps_i, eps_v, W_ie_stem, W_ie_blocks, W_i_mu, W_i_sig,
           W_i_router, b_i_router, W_i_exp1, W_i_exp2, W_ve_stem, W_ve_blocks,
           W_v_mu, W_v_sig, W_v_router, b_v_router, W_v_exp1, W_v_exp2,
           W_f_stem, W_f_blocks, W_f_out):
    lp, mu_l, sg_l = _branch(i, eps_i, W_ie_stem, W_ie_blocks, W_i_mu, W_i_sig,
                             W_i_router, b_i_router, W_i_exp1, W_i_exp2)
    gp, mu_g, sg_g = _branch(v, eps_v, W_ve_stem, W_ve_blocks, W_v_mu, W_v_sig,
                             W_v_router, b_v_router, W_v_exp1, W_v_exp2)

    fusion = _fusion(_to_hcw_pad(v), gp, _to_hcw_pad(i), lp,
                     _prep_w3(W_f_stem, cin_pad=8), _prep_w3(W_f_blocks[0]),
                     _prep_w3(W_f_blocks[1]), _prep_w3(W_f_blocks[2]),
                     _prep_w3(W_f_out))

    return (_from_hcw(fusion), _from_hcw(_unpad(lp)), _from_hcw(_unpad(gp)),
            _from_hcw(mu_l), _from_hcw(sg_l), _from_hcw(mu_g), _from_hcw(sg_g))


# speedup vs baseline: 1.3201x; 1.1226x over previous
import functools

import jax
import jax.numpy as jnp
from jax.experimental import pallas as pl
from jax.experimental.pallas import tpu as pltpu

_PREC = jax.lax.Precision.HIGHEST
_CPREC = jax.lax.Precision.DEFAULT
_F32 = jnp.float32


def _row3(x_ref, y, cin, wp):
    return x_ref[pl.ds(y, 3)].reshape(3 * cin, wp)


def _split_hi_lo(x):
    hi = x.astype(jnp.bfloat16)
    lo = (x - hi.astype(_F32)).astype(jnp.bfloat16)
    return hi, lo


def _stack_w(w3):
    hi = w3.astype(jnp.bfloat16)
    lo = (w3 - hi.astype(_F32)).astype(jnp.bfloat16)
    return jnp.concatenate([hi, lo], axis=-2)


def _dot3s(w2, b_hi, b_lo):
    cout = w2.shape[0] // 2
    s = jnp.dot(w2, b_hi, preferred_element_type=_F32)
    t = jnp.dot(w2[:cout], b_lo, preferred_element_type=_F32)
    return s[:cout] + s[cout:] + t


def _dot3_taps(wall, whi3, b_hi, b_lo):
    co = whi3.shape[0] // 3
    s = jnp.dot(wall, b_hi, preferred_element_type=_F32)
    t = jnp.dot(whi3, b_lo, preferred_element_type=_F32)

    def p(dx):
        return (s[2 * co * dx:2 * co * dx + co]
                + s[2 * co * dx + co:2 * co * (dx + 1)]
                + t[co * dx:co * (dx + 1)])

    return p(0), p(1), p(2)


def _lane_mask(wp):
    lane = jax.lax.broadcasted_iota(jnp.int32, (1, wp), 1)
    return (lane >= 1) & (lane < wp - 1)


def _conv_row(wfun, x_hi, x_lo, wp, pad_out, mask):
    wall, whi3 = wfun()
    p0, p1, p2 = _dot3_taps(wall, whi3, x_hi, x_lo)
    cout = p0.shape[0]
    if pad_out:
        zc = jnp.zeros((cout, 1), _F32)
        s = (jnp.concatenate([zc, p0[:, :wp - 1]], axis=1) + p1
             + jnp.concatenate([p2[:, 1:], zc], axis=1))
        return s, mask
    return p0[:, 0:wp - 2] + p1[:, 1:wp - 1] + p2[:, 2:wp], None


def _pad_row(acc):
    cout = acc.shape[0]
    zcol = jnp.zeros((cout, 1), _F32)
    return jnp.concatenate([zcol, acc, zcol], axis=1)


_UNROLL = 16


def _load3_hilo(src, y, wp):
    if isinstance(src, tuple):
        cin = src[0].shape[1]
        return (_row3(src[0], y, cin, wp), _row3(src[1], y, cin, wp))
    return _split_hi_lo(_row3(src, y, src.shape[1], wp))


def _store_row(dst, y, s):
    if isinstance(dst, tuple):
        sh, sl = _split_hi_lo(s)
        dst[0][y] = sh
        dst[1][y] = sl
    else:
        dst[y] = s


def _zero_row(dst, y, cout, wp):
    if isinstance(dst, tuple):
        dst[0][y] = jnp.zeros((cout, wp), jnp.bfloat16)
        dst[1][y] = jnp.zeros((cout, wp), jnp.bfloat16)
    else:
        dst[y] = jnp.zeros((cout, wp), _F32)


def _conv_pass(src, dst, wfun, relu, pad_out, hh, ww, mask):
    cout = (dst[0] if isinstance(dst, tuple) else dst).shape[1]
    if pad_out:
        _zero_row(dst, 0, cout, ww + 2)
        _zero_row(dst, hh + 1, cout, ww + 2)

    def rows(it, carry):
        y0 = it * _UNROLL
        for u in range(_UNROLL):
            y = y0 + u
            x3h, x3l = _load3_hilo(src, y, ww + 2)
            s, m = _conv_row(wfun, x3h, x3l, ww + 2, pad_out, mask)
            if relu:
                s = jnp.maximum(s, 0.0)
            if pad_out:
                _store_row(dst, y + 1, jnp.where(m, s, 0.0))
            else:
                _store_row(dst, y, s)
        return carry

    jax.lax.fori_loop(0, hh // _UNROLL, rows, 0)


def _enc3_kernel(x_ref, w0a, w0h, w1a, w1h, w2a, w2h, o_ref, s1h, s1l, s2h,
                 s2l, *, hh, ww):
    mask = _lane_mask(ww + 2)
    _conv_pass(x_ref, (s1h, s1l), lambda: (w0a[...], w0h[...]), True, True,
               hh, ww, mask)
    _conv_pass((s1h, s1l), (s2h, s2l), lambda: (w1a[...], w1h[...]), True,
               True, hh, ww, mask)
    _conv_pass((s2h, s2l), o_ref, lambda: (w2a[...], w2h[...]), True, True,
               hh, ww, mask)


def _enc3(xp, w0, w1, w2):
    hp, _, wp = xp.shape
    hh, ww = hp - 2, wp - 2
    c = w0[1].shape[0] // 3
    bf = jnp.bfloat16
    return pl.pallas_call(
        functools.partial(_enc3_kernel, hh=hh, ww=ww),
        out_shape=jax.ShapeDtypeStruct((hp, c, wp), _F32),
        scratch_shapes=[pltpu.VMEM((hp, c, wp), bf),
                        pltpu.VMEM((hp, c, wp), bf),
                        pltpu.VMEM((hp, c, wp), bf),
                        pltpu.VMEM((hp, c, wp), bf)],
    )(xp, w0[0], w0[1], w1[0], w1[1], w2[0], w2[1])


def _fusion_kernel(v_ref, g_ref, i_ref, l_ref, wsa, wsh, wb0a, wb0h, wb1a,
                   wb1h, wb2a, wb2h, woa, woh, o_ref, f_ref, s1h, s1l, s2h,
                   s2l, *, hh, ww):
    cpad = f_ref.shape[1] - v_ref.shape[1]
    wp = ww + 2
    mask = _lane_mask(wp)

    def addrows(it, carry):
        y0 = it * 2
        for u in range(2):
            y = y0 + u
            s = v_ref[y] + g_ref[y] + (i_ref[y] + l_ref[y])
            f_ref[y] = jnp.concatenate([s, jnp.zeros((cpad, wp), _F32)],
                                       axis=0)
        return carry

    jax.lax.fori_loop(0, (hh + 2) // 2, addrows, 0)

    s1 = (s1h, s1l)
    s2 = (s2h, s2l)
    _conv_pass(f_ref, s1, lambda: (wsa[...], wsh[...]), True, True, hh, ww,
               mask)
    _conv_pass(s1, s2, lambda: (wb0a[...], wb0h[...]), True, True, hh, ww,
               mask)
    _conv_pass(s2, s1, lambda: (wb1a[...], wb1h[...]), True, True, hh, ww,
               mask)
    _conv_pass(s1, s2, lambda: (wb2a[...], wb2h[...]), True, True, hh, ww,
               mask)
    _conv_pass(s2, o_ref, lambda: (woa[...], woh[...]), False, False, hh, ww,
               mask)


def _fusion(vp, gp, ip, lp, ws, wb0, wb1, wb2, wo):
    hp, _, wp = vp.shape
    hh, ww = hp - 2, wp - 2
    c = ws[1].shape[0] // 3
    cout = wo[1].shape[0] // 3
    bf = jnp.bfloat16
    return pl.pallas_call(
        functools.partial(_fusion_kernel, hh=hh, ww=ww),
        out_shape=jax.ShapeDtypeStruct((hh, cout, ww), _F32),
        scratch_shapes=[pltpu.VMEM((hp, 8, wp), _F32),
                        pltpu.VMEM((hp, c, wp), bf),
                        pltpu.VMEM((hp, c, wp), bf),
                        pltpu.VMEM((hp, c, wp), bf),
                        pltpu.VMEM((hp, c, wp), bf)],
    )(vp, gp, ip, lp, ws[0], ws[1], wb0[0], wb0[1], wb1[0], wb1[1], wb2[0],
      wb2[1], wo[0], wo[1])


def _musig_kernel(h_ref, wms_ref, mu_ref, sg_ref, *, hh, ww):
    c = mu_ref.shape[1]

    def rows(it, carry):
        y0 = it * _UNROLL
        for u in range(_UNROLL):
            y = y0 + u
            hrow = h_ref[y + 1, :, 1:ww + 1]
            h_hi, h_lo = _split_hi_lo(hrow)
            ms = _dot3s(wms_ref[...], h_hi, h_lo)
            mu_ref[y] = ms[:c]
            raw = ms[c:]
            sp = jnp.maximum(raw, 0.0) + jnp.log1p(jnp.exp(-jnp.abs(raw)))
            sg_ref[y] = sp + 1e-6
        return carry

    jax.lax.fori_loop(0, hh // _UNROLL, rows, 0)


def _musig(hp, wms):
    hp_, c4 = hp.shape[0], wms.shape[0]
    hh, ww = hp_ - 2, hp.shape[2] - 2
    c = c4 // 4
    return pl.pallas_call(
        functools.partial(_musig_kernel, hh=hh, ww=ww),
        out_shape=(jax.ShapeDtypeStruct((hh, c, ww), _F32),
                   jax.ShapeDtypeStruct((hh, c, ww), _F32)),
    )(hp, wms)


def _viz_kernel(mu_ref, sg_ref, eps_ref, wr_ref, bm_ref, zh_ref, zl_ref,
                rt_ref, acc_ref, *, hh, ww):
    c = mu_ref.shape[1]
    _zero_row((zh_ref, zl_ref), 0, c, ww + 2)
    _zero_row((zh_ref, zl_ref), hh + 1, c, ww + 2)
    acc_ref[...] = jnp.zeros((c, ww), _F32)

    def rows(it, carry):
        y0 = it * _UNROLL
        acc = acc_ref[...]
        for u in range(_UNROLL):
            y = y0 + u
            z = mu_ref[y] + jnp.sqrt(sg_ref[y]) * eps_ref[y]
            _store_row((zh_ref, zl_ref), y + 1, _pad_row(z))
            acc = acc + z
        acc_ref[...] = acc
        return carry

    jax.lax.fori_loop(0, hh // _UNROLL, rows, 0)

    pooled_mat = jnp.dot(wr_ref[...], acc_ref[...], precision=_PREC)
    logits = jnp.sum(pooled_mat, axis=1, keepdims=True) / (hh * ww)
    logits = logits + bm_ref[...]
    sub = jax.lax.broadcasted_iota(jnp.int32, (8, 1), 0)
    neg = jnp.float32(-1e30)
    m1 = jnp.max(logits)
    i1 = -jnp.max(jnp.where(logits == m1, -sub.astype(_F32), neg))
    masked = jnp.where(sub.astype(_F32) == i1, neg, logits)
    m2 = jnp.max(masked)
    i2 = -jnp.max(jnp.where(masked == m2, -sub.astype(_F32), neg))
    e1 = jnp.exp(m1 - m1)
    e2 = jnp.exp(m2 - m1)
    g1 = e1 / (e1 + e2)
    g2 = e2 / (e1 + e2)
    out = jnp.where(sub == 0, i1,
          jnp.where(sub == 1, i2,
          jnp.where(sub == 2, g1,
          jnp.where(sub == 3, g2, 0.0))))
    rt_ref[...] = jnp.broadcast_to(out, (8, 128))


def _viz(mu, sg, eps, wr, bmask):
    hh, c, ww = mu.shape
    bf = jnp.bfloat16
    return pl.pallas_call(
        functools.partial(_viz_kernel, hh=hh, ww=ww),
        out_shape=(jax.ShapeDtypeStruct((hh + 2, c, ww + 2), bf),
                   jax.ShapeDtypeStruct((hh + 2, c, ww + 2), bf),
                   jax.ShapeDtypeStruct((8, 128), _F32)),
        scratch_shapes=[pltpu.VMEM((c, ww), _F32)],
    )(mu, sg, eps, wr, bmask)


def _moe_kernel(zh_ref, zl_ref, w1a_ref, w1h_ref, w2a_ref, w2h_ref, idx_ref,
                gate_ref, o_ref, ah_ref, al_ref, *, hh, ww):
    c = zh_ref.shape[1]
    cout = o_ref.shape[1]
    o_ref[0] = jnp.zeros((cout, ww + 2), _F32)
    o_ref[hh + 1] = jnp.zeros((cout, ww + 2), _F32)
    a = (ah_ref, al_ref)
    _zero_row(a, 0, c, ww + 2)
    _zero_row(a, hh + 1, c, ww + 2)
    mask = _lane_mask(ww + 2)

    for k in (0, 1):
        e = idx_ref[k]
        g = gate_ref[k]

        def rows1(it, carry):
            y0 = it * _UNROLL
            for u in range(_UNROLL):
                y = y0 + u
                x3h, x3l = _load3_hilo((zh_ref, zl_ref), y, ww + 2)
                s, m = _conv_row(lambda: (w1a_ref[e], w1h_ref[e]), x3h, x3l,
                                 ww + 2, True, mask)
                _store_row(a, y + 1,
                           jnp.where(m, jnp.maximum(s, 0.0), 0.0))
            return carry

        jax.lax.fori_loop(0, hh // _UNROLL, rows1, 0)

        def rows2(it, carry):
            y0 = it * _UNROLL
            for u in range(_UNROLL):
                y = y0 + u
                x3h, x3l = _load3_hilo(a, y, ww + 2)
                s, m = _conv_row(lambda: (w2a_ref[e], w2h_ref[e]), x3h, x3l,
                                 ww + 2, True, mask)
                s = jnp.where(m, s * g, 0.0)
                if k == 0:
                    o_ref[y + 1] = s
                else:
                    o_ref[y + 1] = o_ref[y + 1] + s
            return carry

        jax.lax.fori_loop(0, hh // _UNROLL, rows2, 0)


def _moe(zh, zl, w1, w2, idx, gate):
    hp, c, wp = zh.shape
    hh, ww = hp - 2, wp - 2
    cout = w2[1].shape[1] // 3
    bf = jnp.bfloat16
    vm = pl.BlockSpec(memory_space=pltpu.VMEM)
    sm = pl.BlockSpec(memory_space=pltpu.SMEM)
    return pl.pallas_call(
        functools.partial(_moe_kernel, hh=hh, ww=ww),
        out_shape=jax.ShapeDtypeStruct((hp, cout, wp), _F32),
        in_specs=[vm, vm, vm, vm, vm, vm, sm, sm],
        scratch_shapes=[pltpu.VMEM((hp, c, wp), bf),
                        pltpu.VMEM((hp, c, wp), bf)],
    )(zh, zl, w1[0], w1[1], w2[0], w2[1], idx, gate)


def _prep_w3(w, cin_pad=None):
    cout, cin = w.shape[0], w.shape[1]
    if cin_pad is not None and cin_pad > cin:
        w = jnp.pad(w, ((0, 0), (0, cin_pad - cin), (0, 0), (0, 0)))
        cin = cin_pad
    w3s = _stack_w(jnp.transpose(w, (3, 0, 2, 1)).reshape(3, cout, 3 * cin))
    wall = w3s.reshape(6 * cout, 3 * cin)
    whi3 = w3s[:, :cout, :].reshape(3 * cout, 3 * cin)
    return wall, whi3


def _prep_wexp(w):
    e, cout, cin = w.shape[0], w.shape[1], w.shape[2]
    w3s = _stack_w(
        jnp.transpose(w, (0, 4, 1, 3, 2)).reshape(e, 3, cout, 3 * cin))
    wall = w3s.reshape(e, 6 * cout, 3 * cin)
    whi3 = w3s[:, :, :cout, :].reshape(e, 3 * cout, 3 * cin)
    return wall, whi3


def _to_hcw_pad(x, cpad=None):
    t = jnp.transpose(x[0], (1, 0, 2))
    extra = 0 if cpad is None else max(0, cpad - t.shape[1])
    return jnp.pad(t, ((1, 1), (0, extra), (1, 1)))


def _to_hcw(x):
    return jnp.transpose(x[0], (1, 0, 2))


def _from_hcw(x):
    return jnp.transpose(x, (1, 0, 2))[None]


def _unpad(xp):
    return xp[1:-1, :, 1:-1]


def _branch(x_nchw, eps_nchw, w_stem, w_blocks, w_mu, w_sig, w_router,
            b_router, w_exp1, w_exp2):
    xp = _to_hcw_pad(x_nchw, cpad=8)
    h = _enc3(xp, _prep_w3(w_stem, cin_pad=8), _prep_w3(w_blocks[0]),
              _prep_w3(w_blocks[1]))

    wms = _stack_w(jnp.concatenate([w_mu[:, :, 0, 0], w_sig[:, :, 0, 0]],
                                   axis=0))
    mu, sg2 = _musig(h, wms)

    e = w_router.shape[0]
    wr = jnp.pad(w_router, ((0, 8 - e), (0, 0)))
    bmask = jnp.pad(b_router, (0, 8 - e),
                    constant_values=-1e30).reshape(8, 1).astype(_F32)
    zh, zl, rt = _viz(mu, sg2, _to_hcw(eps_nchw), wr, bmask)

    idx = rt[0:2, 0].astype(jnp.int32)
    gate = rt[2:4, 0]
    dec = _moe(zh, zl, _prep_wexp(w_exp1), _prep_wexp(w_exp2), idx, gate)
    return dec, mu, sg2


def kernel(i, v, eps_i, eps_v, W_ie_stem, W_ie_blocks, W_i_mu, W_i_sig,
           W_i_router, b_i_router, W_i_exp1, W_i_exp2, W_ve_stem, W_ve_blocks,
           W_v_mu, W_v_sig, W_v_router, b_v_router, W_v_exp1, W_v_exp2,
           W_f_stem, W_f_blocks, W_f_out):
    lp, mu_l, sg_l = _branch(i, eps_i, W_ie_stem, W_ie_blocks, W_i_mu, W_i_sig,
                             W_i_router, b_i_router, W_i_exp1, W_i_exp2)
    gp, mu_g, sg_g = _branch(v, eps_v, W_ve_stem, W_ve_blocks, W_v_mu, W_v_sig,
                             W_v_router, b_v_router, W_v_exp1, W_v_exp2)

    fusion = _fusion(_to_hcw_pad(v), gp, _to_hcw_pad(i), lp,
                     _prep_w3(W_f_stem, cin_pad=8), _prep_w3(W_f_blocks[0]),
                     _prep_w3(W_f_blocks[1]), _prep_w3(W_f_blocks[2]),
                     _prep_w3(W_f_out))

    return (_from_hcw(fusion), _from_hcw(_unpad(lp)), _from_hcw(_unpad(gp)),
            _from_hcw(mu_l), _from_hcw(sg_l), _from_hcw(mu_g), _from_hcw(sg_g))


# speedup vs baseline: 1.4110x; 1.0688x over previous
import functools

import jax
import jax.numpy as jnp
from jax.experimental import pallas as pl
from jax.experimental.pallas import tpu as pltpu

_PREC = jax.lax.Precision.HIGHEST
_CPREC = jax.lax.Precision.DEFAULT
_F32 = jnp.float32


def _row3(x_ref, y, cin, wp):
    return x_ref[pl.ds(y, 3)].reshape(3 * cin, wp)


def _split_hi_lo(x):
    hi = x.astype(jnp.bfloat16)
    lo = (x - hi.astype(_F32)).astype(jnp.bfloat16)
    return hi, lo


def _stack_w(w3):
    hi = w3.astype(jnp.bfloat16)
    lo = (w3 - hi.astype(_F32)).astype(jnp.bfloat16)
    return jnp.concatenate([hi, lo], axis=-2)


def _dot3s(w2, b_hi, b_lo):
    cout = w2.shape[0] // 2
    s = jnp.dot(w2, b_hi, preferred_element_type=_F32)
    t = jnp.dot(w2[:cout], b_lo, preferred_element_type=_F32)
    return s[:cout] + s[cout:] + t


def _dot3_taps(wall, whi3, b_hi, b_lo):
    co = whi3.shape[0] // 3
    s = jnp.dot(wall, b_hi, preferred_element_type=_F32)
    t = jnp.dot(whi3, b_lo, preferred_element_type=_F32)

    def p(dx):
        return (s[2 * co * dx:2 * co * dx + co]
                + s[2 * co * dx + co:2 * co * (dx + 1)]
                + t[co * dx:co * (dx + 1)])

    return p(0), p(1), p(2)


def _lane_mask(wp):
    lane = jax.lax.broadcasted_iota(jnp.int32, (1, wp), 1)
    return (lane >= 1) & (lane < wp - 1)


def _conv_row(wfun, x_hi, x_lo, wp, pad_out, mask):
    wall, whi3 = wfun()
    p0, p1, p2 = _dot3_taps(wall, whi3, x_hi, x_lo)
    cout = p0.shape[0]
    if pad_out:
        zc = jnp.zeros((cout, 1), _F32)
        s = (jnp.concatenate([zc, p0[:, :wp - 1]], axis=1) + p1
             + jnp.concatenate([p2[:, 1:], zc], axis=1))
        return s, mask
    return p0[:, 0:wp - 2] + p1[:, 1:wp - 1] + p2[:, 2:wp], None


def _pad_row(acc):
    cout = acc.shape[0]
    zcol = jnp.zeros((cout, 1), _F32)
    return jnp.concatenate([zcol, acc, zcol], axis=1)


_UNROLL = 32


def _load3_hilo(src, y, wp):
    if isinstance(src, tuple):
        cin = src[0].shape[1]
        return (_row3(src[0], y, cin, wp), _row3(src[1], y, cin, wp))
    return _split_hi_lo(_row3(src, y, src.shape[1], wp))


def _store_row(dst, y, s):
    if isinstance(dst, tuple):
        sh, sl = _split_hi_lo(s)
        dst[0][y] = sh
        dst[1][y] = sl
    else:
        dst[y] = s


def _zero_row(dst, y, cout, wp):
    if isinstance(dst, tuple):
        dst[0][y] = jnp.zeros((cout, wp), jnp.bfloat16)
        dst[1][y] = jnp.zeros((cout, wp), jnp.bfloat16)
    else:
        dst[y] = jnp.zeros((cout, wp), _F32)


def _conv_pass(src, dst, wfun, relu, pad_out, hh, ww, mask):
    cout = (dst[0] if isinstance(dst, tuple) else dst).shape[1]
    if pad_out:
        _zero_row(dst, 0, cout, ww + 2)
        _zero_row(dst, hh + 1, cout, ww + 2)

    def rows(it, carry):
        y0 = it * _UNROLL
        for u in range(_UNROLL):
            y = y0 + u
            x3h, x3l = _load3_hilo(src, y, ww + 2)
            s, m = _conv_row(wfun, x3h, x3l, ww + 2, pad_out, mask)
            if relu:
                s = jnp.maximum(s, 0.0)
            if pad_out:
                _store_row(dst, y + 1, jnp.where(m, s, 0.0))
            else:
                _store_row(dst, y, s)
        return carry

    jax.lax.fori_loop(0, hh // _UNROLL, rows, 0)


def _enc3_kernel(x_ref, w0a, w0h, w1a, w1h, w2a, w2h, o_ref, s1h, s1l, s2h,
                 s2l, *, hh, ww):
    mask = _lane_mask(ww + 2)
    _conv_pass(x_ref, (s1h, s1l), lambda: (w0a[...], w0h[...]), True, True,
               hh, ww, mask)
    _conv_pass((s1h, s1l), (s2h, s2l), lambda: (w1a[...], w1h[...]), True,
               True, hh, ww, mask)
    _conv_pass((s2h, s2l), o_ref, lambda: (w2a[...], w2h[...]), True, True,
               hh, ww, mask)


def _enc3(xp, w0, w1, w2):
    hp, _, wp = xp.shape
    hh, ww = hp - 2, wp - 2
    c = w0[1].shape[0] // 3
    bf = jnp.bfloat16
    return pl.pallas_call(
        functools.partial(_enc3_kernel, hh=hh, ww=ww),
        out_shape=jax.ShapeDtypeStruct((hp, c, wp), _F32),
        scratch_shapes=[pltpu.VMEM((hp, c, wp), bf),
                        pltpu.VMEM((hp, c, wp), bf),
                        pltpu.VMEM((hp, c, wp), bf),
                        pltpu.VMEM((hp, c, wp), bf)],
    )(xp, w0[0], w0[1], w1[0], w1[1], w2[0], w2[1])


def _fusion_kernel(v_ref, g_ref, i_ref, l_ref, wsa, wsh, wb0a, wb0h, wb1a,
                   wb1h, wb2a, wb2h, woa, woh, o_ref, f_ref, s1h, s1l, s2h,
                   s2l, *, hh, ww):
    cpad = f_ref.shape[1] - v_ref.shape[1]
    wp = ww + 2
    mask = _lane_mask(wp)

    def addrows(it, carry):
        y0 = it * 2
        for u in range(2):
            y = y0 + u
            s = v_ref[y] + g_ref[y] + (i_ref[y] + l_ref[y])
            f_ref[y] = jnp.concatenate([s, jnp.zeros((cpad, wp), _F32)],
                                       axis=0)
        return carry

    jax.lax.fori_loop(0, (hh + 2) // 2, addrows, 0)

    s1 = (s1h, s1l)
    s2 = (s2h, s2l)
    _conv_pass(f_ref, s1, lambda: (wsa[...], wsh[...]), True, True, hh, ww,
               mask)
    _conv_pass(s1, s2, lambda: (wb0a[...], wb0h[...]), True, True, hh, ww,
               mask)
    _conv_pass(s2, s1, lambda: (wb1a[...], wb1h[...]), True, True, hh, ww,
               mask)
    _conv_pass(s1, s2, lambda: (wb2a[...], wb2h[...]), True, True, hh, ww,
               mask)
    _conv_pass(s2, o_ref, lambda: (woa[...], woh[...]), False, False, hh, ww,
               mask)


def _fusion(vp, gp, ip, lp, ws, wb0, wb1, wb2, wo):
    hp, _, wp = vp.shape
    hh, ww = hp - 2, wp - 2
    c = ws[1].shape[0] // 3
    cout = wo[1].shape[0] // 3
    bf = jnp.bfloat16
    return pl.pallas_call(
        functools.partial(_fusion_kernel, hh=hh, ww=ww),
        out_shape=jax.ShapeDtypeStruct((hh, cout, ww), _F32),
        scratch_shapes=[pltpu.VMEM((hp, 8, wp), _F32),
                        pltpu.VMEM((hp, c, wp), bf),
                        pltpu.VMEM((hp, c, wp), bf),
                        pltpu.VMEM((hp, c, wp), bf),
                        pltpu.VMEM((hp, c, wp), bf)],
    )(vp, gp, ip, lp, ws[0], ws[1], wb0[0], wb0[1], wb1[0], wb1[1], wb2[0],
      wb2[1], wo[0], wo[1])


def _musig_kernel(h_ref, wms_ref, mu_ref, sg_ref, *, hh, ww):
    c = mu_ref.shape[1]

    def rows(it, carry):
        y0 = it * _UNROLL
        for u in range(_UNROLL):
            y = y0 + u
            hrow = h_ref[y + 1, :, 1:ww + 1]
            h_hi, h_lo = _split_hi_lo(hrow)
            ms = _dot3s(wms_ref[...], h_hi, h_lo)
            mu_ref[y] = ms[:c]
            raw = ms[c:]
            sp = jnp.maximum(raw, 0.0) + jnp.log1p(jnp.exp(-jnp.abs(raw)))
            sg_ref[y] = sp + 1e-6
        return carry

    jax.lax.fori_loop(0, hh // _UNROLL, rows, 0)


def _musig(hp, wms):
    hp_, c4 = hp.shape[0], wms.shape[0]
    hh, ww = hp_ - 2, hp.shape[2] - 2
    c = c4 // 4
    return pl.pallas_call(
        functools.partial(_musig_kernel, hh=hh, ww=ww),
        out_shape=(jax.ShapeDtypeStruct((hh, c, ww), _F32),
                   jax.ShapeDtypeStruct((hh, c, ww), _F32)),
    )(hp, wms)


def _viz_kernel(mu_ref, sg_ref, eps_ref, wr_ref, bm_ref, zh_ref, zl_ref,
                rt_ref, acc_ref, *, hh, ww):
    c = mu_ref.shape[1]
    _zero_row((zh_ref, zl_ref), 0, c, ww + 2)
    _zero_row((zh_ref, zl_ref), hh + 1, c, ww + 2)
    acc_ref[...] = jnp.zeros((c, ww), _F32)

    def rows(it, carry):
        y0 = it * _UNROLL
        acc = acc_ref[...]
        for u in range(_UNROLL):
            y = y0 + u
            z = mu_ref[y] + jnp.sqrt(sg_ref[y]) * eps_ref[y]
            _store_row((zh_ref, zl_ref), y + 1, _pad_row(z))
            acc = acc + z
        acc_ref[...] = acc
        return carry

    jax.lax.fori_loop(0, hh // _UNROLL, rows, 0)

    pooled_mat = jnp.dot(wr_ref[...], acc_ref[...], precision=_PREC)
    logits = jnp.sum(pooled_mat, axis=1, keepdims=True) / (hh * ww)
    logits = logits + bm_ref[...]
    sub = jax.lax.broadcasted_iota(jnp.int32, (8, 1), 0)
    neg = jnp.float32(-1e30)
    m1 = jnp.max(logits)
    i1 = -jnp.max(jnp.where(logits == m1, -sub.astype(_F32), neg))
    masked = jnp.where(sub.astype(_F32) == i1, neg, logits)
    m2 = jnp.max(masked)
    i2 = -jnp.max(jnp.where(masked == m2, -sub.astype(_F32), neg))
    e1 = jnp.exp(m1 - m1)
    e2 = jnp.exp(m2 - m1)
    g1 = e1 / (e1 + e2)
    g2 = e2 / (e1 + e2)
    out = jnp.where(sub == 0, i1,
          jnp.where(sub == 1, i2,
          jnp.where(sub == 2, g1,
          jnp.where(sub == 3, g2, 0.0))))
    rt_ref[...] = jnp.broadcast_to(out, (8, 128))


def _viz(mu, sg, eps, wr, bmask):
    hh, c, ww = mu.shape
    bf = jnp.bfloat16
    return pl.pallas_call(
        functools.partial(_viz_kernel, hh=hh, ww=ww),
        out_shape=(jax.ShapeDtypeStruct((hh + 2, c, ww + 2), bf),
                   jax.ShapeDtypeStruct((hh + 2, c, ww + 2), bf),
                   jax.ShapeDtypeStruct((8, 128), _F32)),
        scratch_shapes=[pltpu.VMEM((c, ww), _F32)],
    )(mu, sg, eps, wr, bmask)


def _moe_kernel(zh_ref, zl_ref, w1a_ref, w1h_ref, w2a_ref, w2h_ref, idx_ref,
                gate_ref, o_ref, ah_ref, al_ref, *, hh, ww):
    c = zh_ref.shape[1]
    cout = o_ref.shape[1]
    o_ref[0] = jnp.zeros((cout, ww + 2), _F32)
    o_ref[hh + 1] = jnp.zeros((cout, ww + 2), _F32)
    a = (ah_ref, al_ref)
    _zero_row(a, 0, c, ww + 2)
    _zero_row(a, hh + 1, c, ww + 2)
    mask = _lane_mask(ww + 2)

    for k in (0, 1):
        e = idx_ref[k]
        g = gate_ref[k]

        def rows1(it, carry):
            y0 = it * _UNROLL
            for u in range(_UNROLL):
                y = y0 + u
                x3h, x3l = _load3_hilo((zh_ref, zl_ref), y, ww + 2)
                s, m = _conv_row(lambda: (w1a_ref[e], w1h_ref[e]), x3h, x3l,
                                 ww + 2, True, mask)
                _store_row(a, y + 1,
                           jnp.where(m, jnp.maximum(s, 0.0), 0.0))
            return carry

        jax.lax.fori_loop(0, hh // _UNROLL, rows1, 0)

        def rows2(it, carry):
            y0 = it * _UNROLL
            for u in range(_UNROLL):
                y = y0 + u
                x3h, x3l = _load3_hilo(a, y, ww + 2)
                s, m = _conv_row(lambda: (w2a_ref[e], w2h_ref[e]), x3h, x3l,
                                 ww + 2, True, mask)
                s = jnp.where(m, s * g, 0.0)
                if k == 0:
                    o_ref[y + 1] = s
                else:
                    o_ref[y + 1] = o_ref[y + 1] + s
            return carry

        jax.lax.fori_loop(0, hh // _UNROLL, rows2, 0)


def _moe(zh, zl, w1, w2, idx, gate):
    hp, c, wp = zh.shape
    hh, ww = hp - 2, wp - 2
    cout = w2[1].shape[1] // 3
    bf = jnp.bfloat16
    vm = pl.BlockSpec(memory_space=pltpu.VMEM)
    sm = pl.BlockSpec(memory_space=pltpu.SMEM)
    return pl.pallas_call(
        functools.partial(_moe_kernel, hh=hh, ww=ww),
        out_shape=jax.ShapeDtypeStruct((hp, cout, wp), _F32),
        in_specs=[vm, vm, vm, vm, vm, vm, sm, sm],
        scratch_shapes=[pltpu.VMEM((hp, c, wp), bf),
                        pltpu.VMEM((hp, c, wp), bf)],
    )(zh, zl, w1[0], w1[1], w2[0], w2[1], idx, gate)


def _prep_w3(w, cin_pad=None):
    cout, cin = w.shape[0], w.shape[1]
    if cin_pad is not None and cin_pad > cin:
        w = jnp.pad(w, ((0, 0), (0, cin_pad - cin), (0, 0), (0, 0)))
        cin = cin_pad
    w3s = _stack_w(jnp.transpose(w, (3, 0, 2, 1)).reshape(3, cout, 3 * cin))
    wall = w3s.reshape(6 * cout, 3 * cin)
    whi3 = w3s[:, :cout, :].reshape(3 * cout, 3 * cin)
    return wall, whi3


def _prep_wexp(w):
    e, cout, cin = w.shape[0], w.shape[1], w.shape[2]
    w3s = _stack_w(
        jnp.transpose(w, (0, 4, 1, 3, 2)).reshape(e, 3, cout, 3 * cin))
    wall = w3s.reshape(e, 6 * cout, 3 * cin)
    whi3 = w3s[:, :, :cout, :].reshape(e, 3 * cout, 3 * cin)
    return wall, whi3


def _to_hcw_pad(x, cpad=None):
    t = jnp.transpose(x[0], (1, 0, 2))
    extra = 0 if cpad is None else max(0, cpad - t.shape[1])
    return jnp.pad(t, ((1, 1), (0, extra), (1, 1)))


def _to_hcw(x):
    return jnp.transpose(x[0], (1, 0, 2))


def _from_hcw(x):
    return jnp.transpose(x, (1, 0, 2))[None]


def _unpad(xp):
    return xp[1:-1, :, 1:-1]


def _branch(x_nchw, eps_nchw, w_stem, w_blocks, w_mu, w_sig, w_router,
            b_router, w_exp1, w_exp2):
    xp = _to_hcw_pad(x_nchw, cpad=8)
    h = _enc3(xp, _prep_w3(w_stem, cin_pad=8), _prep_w3(w_blocks[0]),
              _prep_w3(w_blocks[1]))

    wms = _stack_w(jnp.concatenate([w_mu[:, :, 0, 0], w_sig[:, :, 0, 0]],
                                   axis=0))
    mu, sg2 = _musig(h, wms)

    e = w_router.shape[0]
    wr = jnp.pad(w_router, ((0, 8 - e), (0, 0)))
    bmask = jnp.pad(b_router, (0, 8 - e),
                    constant_values=-1e30).reshape(8, 1).astype(_F32)
    zh, zl, rt = _viz(mu, sg2, _to_hcw(eps_nchw), wr, bmask)

    idx = rt[0:2, 0].astype(jnp.int32)
    gate = rt[2:4, 0]
    dec = _moe(zh, zl, _prep_wexp(w_exp1), _prep_wexp(w_exp2), idx, gate)
    return dec, mu, sg2


def kernel(i, v, eps_i, eps_v, W_ie_stem, W_ie_blocks, W_i_mu, W_i_sig,
           W_i_router, b_i_router, W_i_exp1, W_i_exp2, W_ve_stem, W_ve_blocks,
           W_v_mu, W_v_sig, W_v_router, b_v_router, W_v_exp1, W_v_exp2,
           W_f_stem, W_f_blocks, W_f_out):
    lp, mu_l, sg_l = _branch(i, eps_i, W_ie_stem, W_ie_blocks, W_i_mu, W_i_sig,
                             W_i_router, b_i_router, W_i_exp1, W_i_exp2)
    gp, mu_g, sg_g = _branch(v, eps_v, W_ve_stem, W_ve_blocks, W_v_mu, W_v_sig,
                             W_v_router, b_v_router, W_v_exp1, W_v_exp2)

    fusion = _fusion(_to_hcw_pad(v), gp, _to_hcw_pad(i), lp,
                     _prep_w3(W_f_stem, cin_pad=8), _prep_w3(W_f_blocks[0]),
                     _prep_w3(W_f_blocks[1]), _prep_w3(W_f_blocks[2]),
                     _prep_w3(W_f_out))

    return (_from_hcw(fusion), _from_hcw(_unpad(lp)), _from_hcw(_unpad(gp)),
            _from_hcw(mu_l), _from_hcw(sg_l), _from_hcw(mu_g), _from_hcw(sg_g))


# speedup vs baseline: 1.4463x; 1.0250x over previous
import functools

import jax
import jax.numpy as jnp
from jax.experimental import pallas as pl
from jax.experimental.pallas import tpu as pltpu

_PREC = jax.lax.Precision.HIGHEST
_CPREC = jax.lax.Precision.DEFAULT
_F32 = jnp.float32


def _row3(x_ref, y, cin, wp):
    return x_ref[pl.ds(y, 3)].reshape(3 * cin, wp)


def _split_hi_lo(x):
    hi = x.astype(jnp.bfloat16)
    lo = (x - hi.astype(_F32)).astype(jnp.bfloat16)
    return hi, lo


def _stack_w(w3):
    hi = w3.astype(jnp.bfloat16)
    lo = (w3 - hi.astype(_F32)).astype(jnp.bfloat16)
    return jnp.concatenate([hi, lo], axis=-2)


def _dot3s(w2, b_hi, b_lo):
    cout = w2.shape[0] // 2
    s = jnp.dot(w2, b_hi, preferred_element_type=_F32)
    t = jnp.dot(w2[:cout], b_lo, preferred_element_type=_F32)
    return s[:cout] + s[cout:] + t


def _dot3_taps(wall, whi3, b_hi, b_lo):
    co = whi3.shape[0] // 3
    s = jnp.dot(wall, b_hi, preferred_element_type=_F32)
    t = jnp.dot(whi3, b_lo, preferred_element_type=_F32)

    def p(dx):
        return (s[2 * co * dx:2 * co * dx + co]
                + s[2 * co * dx + co:2 * co * (dx + 1)]
                + t[co * dx:co * (dx + 1)])

    return p(0), p(1), p(2)


def _lane_mask(wp):
    lane = jax.lax.broadcasted_iota(jnp.int32, (1, wp), 1)
    return (lane >= 1) & (lane < wp - 1)


def _conv_row(wfun, x_hi, x_lo, wp, pad_out, mask):
    wall, whi3 = wfun()
    p0, p1, p2 = _dot3_taps(wall, whi3, x_hi, x_lo)
    cout = p0.shape[0]
    if pad_out:
        zc = jnp.zeros((cout, 1), _F32)
        s = (jnp.concatenate([zc, p0[:, :wp - 1]], axis=1) + p1
             + jnp.concatenate([p2[:, 1:], zc], axis=1))
        return s, mask
    return p0[:, 0:wp - 2] + p1[:, 1:wp - 1] + p2[:, 2:wp], None


def _pad_row(acc):
    cout = acc.shape[0]
    zcol = jnp.zeros((cout, 1), _F32)
    return jnp.concatenate([zcol, acc, zcol], axis=1)


_UNROLL = 56


def _load3_hilo(src, y, wp):
    if isinstance(src, tuple):
        cin = src[0].shape[1]
        return (_row3(src[0], y, cin, wp), _row3(src[1], y, cin, wp))
    return _split_hi_lo(_row3(src, y, src.shape[1], wp))


def _store_row(dst, y, s):
    if isinstance(dst, tuple):
        sh, sl = _split_hi_lo(s)
        dst[0][y] = sh
        dst[1][y] = sl
    else:
        dst[y] = s


def _zero_row(dst, y, cout, wp):
    if isinstance(dst, tuple):
        dst[0][y] = jnp.zeros((cout, wp), jnp.bfloat16)
        dst[1][y] = jnp.zeros((cout, wp), jnp.bfloat16)
    else:
        dst[y] = jnp.zeros((cout, wp), _F32)


def _conv_pass(src, dst, wfun, relu, pad_out, hh, ww, mask):
    cout = (dst[0] if isinstance(dst, tuple) else dst).shape[1]
    if pad_out:
        _zero_row(dst, 0, cout, ww + 2)
        _zero_row(dst, hh + 1, cout, ww + 2)

    def rows(it, carry):
        y0 = it * _UNROLL
        for u in range(_UNROLL):
            y = y0 + u
            x3h, x3l = _load3_hilo(src, y, ww + 2)
            s, m = _conv_row(wfun, x3h, x3l, ww + 2, pad_out, mask)
            if relu:
                s = jnp.maximum(s, 0.0)
            if pad_out:
                _store_row(dst, y + 1, jnp.where(m, s, 0.0))
            else:
                _store_row(dst, y, s)
        return carry

    jax.lax.fori_loop(0, hh // _UNROLL, rows, 0)


def _enc3_kernel(x_ref, w0a, w0h, w1a, w1h, w2a, w2h, o_ref, s1h, s1l, s2h,
                 s2l, *, hh, ww):
    mask = _lane_mask(ww + 2)
    _conv_pass(x_ref, (s1h, s1l), lambda: (w0a[...], w0h[...]), True, True,
               hh, ww, mask)
    _conv_pass((s1h, s1l), (s2h, s2l), lambda: (w1a[...], w1h[...]), True,
               True, hh, ww, mask)
    _conv_pass((s2h, s2l), o_ref, lambda: (w2a[...], w2h[...]), True, True,
               hh, ww, mask)


def _enc3(xp, w0, w1, w2):
    hp, _, wp = xp.shape
    hh, ww = hp - 2, wp - 2
    c = w0[1].shape[0] // 3
    bf = jnp.bfloat16
    return pl.pallas_call(
        functools.partial(_enc3_kernel, hh=hh, ww=ww),
        out_shape=jax.ShapeDtypeStruct((hp, c, wp), _F32),
        scratch_shapes=[pltpu.VMEM((hp, c, wp), bf),
                        pltpu.VMEM((hp, c, wp), bf),
                        pltpu.VMEM((hp, c, wp), bf),
                        pltpu.VMEM((hp, c, wp), bf)],
    )(xp, w0[0], w0[1], w1[0], w1[1], w2[0], w2[1])


def _fusion_kernel(v_ref, g_ref, i_ref, l_ref, wsa, wsh, wb0a, wb0h, wb1a,
                   wb1h, wb2a, wb2h, woa, woh, o_ref, f_ref, s1h, s1l, s2h,
                   s2l, *, hh, ww):
    cpad = f_ref.shape[1] - v_ref.shape[1]
    wp = ww + 2
    mask = _lane_mask(wp)

    def addrows(it, carry):
        y0 = it * 2
        for u in range(2):
            y = y0 + u
            s = v_ref[y] + g_ref[y] + (i_ref[y] + l_ref[y])
            f_ref[y] = jnp.concatenate([s, jnp.zeros((cpad, wp), _F32)],
                                       axis=0)
        return carry

    jax.lax.fori_loop(0, (hh + 2) // 2, addrows, 0)

    s1 = (s1h, s1l)
    s2 = (s2h, s2l)
    _conv_pass(f_ref, s1, lambda: (wsa[...], wsh[...]), True, True, hh, ww,
               mask)
    _conv_pass(s1, s2, lambda: (wb0a[...], wb0h[...]), True, True, hh, ww,
               mask)
    _conv_pass(s2, s1, lambda: (wb1a[...], wb1h[...]), True, True, hh, ww,
               mask)
    _conv_pass(s1, s2, lambda: (wb2a[...], wb2h[...]), True, True, hh, ww,
               mask)
    _conv_pass(s2, o_ref, lambda: (woa[...], woh[...]), False, False, hh, ww,
               mask)


def _fusion(vp, gp, ip, lp, ws, wb0, wb1, wb2, wo):
    hp, _, wp = vp.shape
    hh, ww = hp - 2, wp - 2
    c = ws[1].shape[0] // 3
    cout = wo[1].shape[0] // 3
    bf = jnp.bfloat16
    return pl.pallas_call(
        functools.partial(_fusion_kernel, hh=hh, ww=ww),
        out_shape=jax.ShapeDtypeStruct((hh, cout, ww), _F32),
        scratch_shapes=[pltpu.VMEM((hp, 8, wp), _F32),
                        pltpu.VMEM((hp, c, wp), bf),
                        pltpu.VMEM((hp, c, wp), bf),
                        pltpu.VMEM((hp, c, wp), bf),
                        pltpu.VMEM((hp, c, wp), bf)],
    )(vp, gp, ip, lp, ws[0], ws[1], wb0[0], wb0[1], wb1[0], wb1[1], wb2[0],
      wb2[1], wo[0], wo[1])


def _musig_kernel(h_ref, wms_ref, mu_ref, sg_ref, *, hh, ww):
    c = mu_ref.shape[1]

    def rows(it, carry):
        y0 = it * _UNROLL
        for u in range(_UNROLL):
            y = y0 + u
            hrow = h_ref[y + 1, :, 1:ww + 1]
            h_hi, h_lo = _split_hi_lo(hrow)
            ms = _dot3s(wms_ref[...], h_hi, h_lo)
            mu_ref[y] = ms[:c]
            raw = ms[c:]
            sp = jnp.maximum(raw, 0.0) + jnp.log1p(jnp.exp(-jnp.abs(raw)))
            sg_ref[y] = sp + 1e-6
        return carry

    jax.lax.fori_loop(0, hh // _UNROLL, rows, 0)


def _musig(hp, wms):
    hp_, c4 = hp.shape[0], wms.shape[0]
    hh, ww = hp_ - 2, hp.shape[2] - 2
    c = c4 // 4
    return pl.pallas_call(
        functools.partial(_musig_kernel, hh=hh, ww=ww),
        out_shape=(jax.ShapeDtypeStruct((hh, c, ww), _F32),
                   jax.ShapeDtypeStruct((hh, c, ww), _F32)),
    )(hp, wms)


def _viz_kernel(mu_ref, sg_ref, eps_ref, wr_ref, bm_ref, zh_ref, zl_ref,
                rt_ref, acc_ref, *, hh, ww):
    c = mu_ref.shape[1]
    _zero_row((zh_ref, zl_ref), 0, c, ww + 2)
    _zero_row((zh_ref, zl_ref), hh + 1, c, ww + 2)
    acc_ref[...] = jnp.zeros((c, ww), _F32)

    def rows(it, carry):
        y0 = it * _UNROLL
        acc = acc_ref[...]
        for u in range(_UNROLL):
            y = y0 + u
            z = mu_ref[y] + jnp.sqrt(sg_ref[y]) * eps_ref[y]
            _store_row((zh_ref, zl_ref), y + 1, _pad_row(z))
            acc = acc + z
        acc_ref[...] = acc
        return carry

    jax.lax.fori_loop(0, hh // _UNROLL, rows, 0)

    pooled_mat = jnp.dot(wr_ref[...], acc_ref[...], precision=_PREC)
    logits = jnp.sum(pooled_mat, axis=1, keepdims=True) / (hh * ww)
    logits = logits + bm_ref[...]
    sub = jax.lax.broadcasted_iota(jnp.int32, (8, 1), 0)
    neg = jnp.float32(-1e30)
    m1 = jnp.max(logits)
    i1 = -jnp.max(jnp.where(logits == m1, -sub.astype(_F32), neg))
    masked = jnp.where(sub.astype(_F32) == i1, neg, logits)
    m2 = jnp.max(masked)
    i2 = -jnp.max(jnp.where(masked == m2, -sub.astype(_F32), neg))
    e1 = jnp.exp(m1 - m1)
    e2 = jnp.exp(m2 - m1)
    g1 = e1 / (e1 + e2)
    g2 = e2 / (e1 + e2)
    out = jnp.where(sub == 0, i1,
          jnp.where(sub == 1, i2,
          jnp.where(sub == 2, g1,
          jnp.where(sub == 3, g2, 0.0))))
    rt_ref[...] = jnp.broadcast_to(out, (8, 128))


def _viz(mu, sg, eps, wr, bmask):
    hh, c, ww = mu.shape
    bf = jnp.bfloat16
    return pl.pallas_call(
        functools.partial(_viz_kernel, hh=hh, ww=ww),
        out_shape=(jax.ShapeDtypeStruct((hh + 2, c, ww + 2), bf),
                   jax.ShapeDtypeStruct((hh + 2, c, ww + 2), bf),
                   jax.ShapeDtypeStruct((8, 128), _F32)),
        scratch_shapes=[pltpu.VMEM((c, ww), _F32)],
    )(mu, sg, eps, wr, bmask)


def _moe_kernel(zh_ref, zl_ref, w1a_ref, w1h_ref, w2a_ref, w2h_ref, idx_ref,
                gate_ref, o_ref, ah_ref, al_ref, *, hh, ww):
    c = zh_ref.shape[1]
    cout = o_ref.shape[1]
    o_ref[0] = jnp.zeros((cout, ww + 2), _F32)
    o_ref[hh + 1] = jnp.zeros((cout, ww + 2), _F32)
    a = (ah_ref, al_ref)
    _zero_row(a, 0, c, ww + 2)
    _zero_row(a, hh + 1, c, ww + 2)
    mask = _lane_mask(ww + 2)

    for k in (0, 1):
        e = idx_ref[k]
        g = gate_ref[k]

        def rows1(it, carry):
            y0 = it * _UNROLL
            for u in range(_UNROLL):
                y = y0 + u
                x3h, x3l = _load3_hilo((zh_ref, zl_ref), y, ww + 2)
                s, m = _conv_row(lambda: (w1a_ref[e], w1h_ref[e]), x3h, x3l,
                                 ww + 2, True, mask)
                _store_row(a, y + 1,
                           jnp.where(m, jnp.maximum(s, 0.0), 0.0))
            return carry

        jax.lax.fori_loop(0, hh // _UNROLL, rows1, 0)

        def rows2(it, carry):
            y0 = it * _UNROLL
            for u in range(_UNROLL):
                y = y0 + u
                x3h, x3l = _load3_hilo(a, y, ww + 2)
                s, m = _conv_row(lambda: (w2a_ref[e], w2h_ref[e]), x3h, x3l,
                                 ww + 2, True, mask)
                s = jnp.where(m, s * g, 0.0)
                if k == 0:
                    o_ref[y + 1] = s
                else:
                    o_ref[y + 1] = o_ref[y + 1] + s
            return carry

        jax.lax.fori_loop(0, hh // _UNROLL, rows2, 0)


def _moe(zh, zl, w1, w2, idx, gate):
    hp, c, wp = zh.shape
    hh, ww = hp - 2, wp - 2
    cout = w2[1].shape[1] // 3
    bf = jnp.bfloat16
    vm = pl.BlockSpec(memory_space=pltpu.VMEM)
    sm = pl.BlockSpec(memory_space=pltpu.SMEM)
    return pl.pallas_call(
        functools.partial(_moe_kernel, hh=hh, ww=ww),
        out_shape=jax.ShapeDtypeStruct((hp, cout, wp), _F32),
        in_specs=[vm, vm, vm, vm, vm, vm, sm, sm],
        scratch_shapes=[pltpu.VMEM((hp, c, wp), bf),
                        pltpu.VMEM((hp, c, wp), bf)],
    )(zh, zl, w1[0], w1[1], w2[0], w2[1], idx, gate)


def _prep_w3(w, cin_pad=None):
    cout, cin = w.shape[0], w.shape[1]
    if cin_pad is not None and cin_pad > cin:
        w = jnp.pad(w, ((0, 0), (0, cin_pad - cin), (0, 0), (0, 0)))
        cin = cin_pad
    w3s = _stack_w(jnp.transpose(w, (3, 0, 2, 1)).reshape(3, cout, 3 * cin))
    wall = w3s.reshape(6 * cout, 3 * cin)
    whi3 = w3s[:, :cout, :].reshape(3 * cout, 3 * cin)
    return wall, whi3


def _prep_wexp(w):
    e, cout, cin = w.shape[0], w.shape[1], w.shape[2]
    w3s = _stack_w(
        jnp.transpose(w, (0, 4, 1, 3, 2)).reshape(e, 3, cout, 3 * cin))
    wall = w3s.reshape(e, 6 * cout, 3 * cin)
    whi3 = w3s[:, :, :cout, :].reshape(e, 3 * cout, 3 * cin)
    return wall, whi3


def _to_hcw_pad(x, cpad=None):
    t = jnp.transpose(x[0], (1, 0, 2))
    extra = 0 if cpad is None else max(0, cpad - t.shape[1])
    return jnp.pad(t, ((1, 1), (0, extra), (1, 1)))


def _to_hcw(x):
    return jnp.transpose(x[0], (1, 0, 2))


def _from_hcw(x):
    return jnp.transpose(x, (1, 0, 2))[None]


def _unpad(xp):
    return xp[1:-1, :, 1:-1]


def _branch(x_nchw, eps_nchw, w_stem, w_blocks, w_mu, w_sig, w_router,
            b_router, w_exp1, w_exp2):
    xp = _to_hcw_pad(x_nchw, cpad=8)
    h = _enc3(xp, _prep_w3(w_stem, cin_pad=8), _prep_w3(w_blocks[0]),
              _prep_w3(w_blocks[1]))

    wms = _stack_w(jnp.concatenate([w_mu[:, :, 0, 0], w_sig[:, :, 0, 0]],
                                   axis=0))
    mu, sg2 = _musig(h, wms)

    e = w_router.shape[0]
    wr = jnp.pad(w_router, ((0, 8 - e), (0, 0)))
    bmask = jnp.pad(b_router, (0, 8 - e),
                    constant_values=-1e30).reshape(8, 1).astype(_F32)
    zh, zl, rt = _viz(mu, sg2, _to_hcw(eps_nchw), wr, bmask)

    idx = rt[0:2, 0].astype(jnp.int32)
    gate = rt[2:4, 0]
    dec = _moe(zh, zl, _prep_wexp(w_exp1), _prep_wexp(w_exp2), idx, gate)
    return dec, mu, sg2


def kernel(i, v, eps_i, eps_v, W_ie_stem, W_ie_blocks, W_i_mu, W_i_sig,
           W_i_router, b_i_router, W_i_exp1, W_i_exp2, W_ve_stem, W_ve_blocks,
           W_v_mu, W_v_sig, W_v_router, b_v_router, W_v_exp1, W_v_exp2,
           W_f_stem, W_f_blocks, W_f_out):
    lp, mu_l, sg_l = _branch(i, eps_i, W_ie_stem, W_ie_blocks, W_i_mu, W_i_sig,
                             W_i_router, b_i_router, W_i_exp1, W_i_exp2)
    gp, mu_g, sg_g = _branch(v, eps_v, W_ve_stem, W_ve_blocks, W_v_mu, W_v_sig,
                             W_v_router, b_v_router, W_v_exp1, W_v_exp2)

    fusion = _fusion(_to_hcw_pad(v), gp, _to_hcw_pad(i), lp,
                     _prep_w3(W_f_stem, cin_pad=8), _prep_w3(W_f_blocks[0]),
                     _prep_w3(W_f_blocks[1]), _prep_w3(W_f_blocks[2]),
                     _prep_w3(W_f_out))

    return (_from_hcw(fusion), _from_hcw(_unpad(lp)), _from_hcw(_unpad(gp)),
            _from_hcw(mu_l), _from_hcw(sg_l), _from_hcw(mu_g), _from_hcw(sg_g))
